# Initial kernel scaffold; baseline (speedup 1.0000x reference)
import jax
import jax.numpy as jnp
from jax.experimental import pallas as pl


def kernel(X, V, E, H, W1, W2, W3):
    raise NotImplementedError("write your pallas kernel here")


# speedup vs baseline: 4.8238x; 4.8238x over previous
import jax
import jax.numpy as jnp
from jax import lax
from jax.experimental import pallas as pl
from jax.experimental.pallas import tpu as pltpu
from jax.experimental.pallas import tpu_sc as plsc

N_NODES = 10000
NNZ = 320000
N_HEDGES = 5000
NFEAT = 128
C = 256
NQ = 4
CH = C // NQ
C3 = 40

NC, NS, L = 2, 16, 16
NPASS = NQ // NC
NH_PAD = 5008
PAIRS_PER_SUB = NNZ // NS
K = 400
NCHUNK = PAIRS_PER_SUB // K
ZROWS = 64
EROWS = NH_PAD // NS
VROWS = N_NODES // NS

_mesh = plsc.VectorSubcoreMesh(core_axis_name="c", subcore_axis_name="s")
_sc_params = pltpu.CompilerParams(use_tc_tiling_on_sc=False)


def _zero_slices(zsrc, dst_sh, row0, nrows):
    done = 0
    while done < nrows:
        nb = min(ZROWS, nrows - done)
        pltpu.sync_copy(zsrc.at[pl.ds(0, nb)], dst_sh.at[pl.ds(row0 + done, nb)])
        done += nb


def _sc_body(xp_hbm, v_hbm, e_hbm, z_hbm, o_hbm, xv_out,
             vidx, eidx, rows, zblk, nbuf, sbuf, ones,
             cnt_sh, esum_sh, xv_sh, sem):
    c = lax.axis_index("c")
    s = lax.axis_index("s")

    pltpu.sync_copy(z_hbm, zblk)
    pltpu.sync_copy(o_hbm, ones)

    for cs in range(NPASS):
        q = c * NPASS + cs

        _zero_slices(zblk, esum_sh, s * EROWS, EROWS)
        _zero_slices(zblk, xv_sh, s * VROWS, VROWS)
        if cs == 0:
            _zero_slices(zblk.at[:, pl.ds(0, L)], cnt_sh, s * EROWS, EROWS)
        plsc.subcore_barrier()

        @pl.loop(0, NCHUNK)
        def _(t):
            base = s * PAIRS_PER_SUB + t * K
            pltpu.sync_copy(v_hbm.at[pl.ds(base, K)], vidx)
            pltpu.sync_copy(e_hbm.at[pl.ds(base, K)], eidx)
            pltpu.async_copy(xp_hbm.at[q].at[vidx], rows, sem).wait()
            pltpu.sync_copy(rows, esum_sh.at[eidx], add=True)
            if cs == 0:
                pltpu.sync_copy(ones, cnt_sh.at[eidx], add=True)

        plsc.subcore_barrier()

        row0 = s * EROWS
        done = 0
        while done < EROWS:
            nb = min(ZROWS, EROWS - done)
            off = row0 + done
            pltpu.sync_copy(esum_sh.at[pl.ds(off, nb)], nbuf.at[pl.ds(0, nb)])
            pltpu.sync_copy(cnt_sh.at[pl.ds(off, nb)], sbuf.at[pl.ds(0, nb)])

            @pl.loop(0, nb)
            def _(r):
                scale = 1.0 / jnp.maximum(sbuf[r, pl.ds(0, L)], 1.0)
                for j in range(CH // L):
                    nbuf[r, pl.ds(j * L, L)] = nbuf[r, pl.ds(j * L, L)] * scale

            pltpu.sync_copy(nbuf.at[pl.ds(0, nb)], esum_sh.at[pl.ds(off, nb)])
            done += nb
        plsc.subcore_barrier()

        @pl.loop(0, NCHUNK)
        def _(t):
            base = s * PAIRS_PER_SUB + t * K
            pltpu.sync_copy(v_hbm.at[pl.ds(base, K)], vidx)
            pltpu.sync_copy(e_hbm.at[pl.ds(base, K)], eidx)
            pltpu.async_copy(esum_sh.at[eidx], rows, sem).wait()
            pltpu.sync_copy(rows, xv_sh.at[vidx], add=True)

        plsc.subcore_barrier()

        pltpu.sync_copy(xv_sh.at[pl.ds(s * VROWS, VROWS)],
                        xv_out.at[q].at[pl.ds(s * VROWS, VROWS)])


_sc_sparse = pl.kernel(
    _sc_body,
    out_type=jax.ShapeDtypeStruct((NQ, N_NODES, CH), jnp.float32),
    mesh=_mesh,
    scratch_types=[
        pltpu.VMEM((K,), jnp.int32),
        pltpu.VMEM((K,), jnp.int32),
        pltpu.VMEM((K, CH), jnp.float32),
        pltpu.VMEM((ZROWS, CH), jnp.float32),
        pltpu.VMEM((ZROWS, CH), jnp.float32),
        pltpu.VMEM((ZROWS, L), jnp.float32),
        pltpu.VMEM((K, L), jnp.float32),
        pltpu.VMEM_SHARED((NH_PAD, L), jnp.float32),
        pltpu.VMEM_SHARED((NH_PAD, CH), jnp.float32),
        pltpu.VMEM_SHARED((N_NODES, CH), jnp.float32),
        pltpu.SemaphoreType.DMA,
    ],
    compiler_params=_sc_params,
)


_RB = 2000
_PREC = jax.lax.Precision.HIGHEST


def _mmf_body(xp_ref, xv_ref, w_ref, c_ref, o_ref):
    a = jnp.concatenate(
        [xp_ref[i] + xv_ref[i] for i in range(NQ)], axis=1)
    a = jnp.maximum(a, 0.0) + jnp.minimum(a, 0.0) * c_ref[...]
    r = jnp.dot(a, w_ref[...], preferred_element_type=jnp.float32,
                precision=_PREC)
    for i in range(NQ):
        o_ref[i] = r[:, i * CH:(i + 1) * CH]


def _mmf(xp, xv, w, slope):
    return pl.pallas_call(
        _mmf_body,
        grid=(N_NODES // _RB,),
        in_specs=[pl.BlockSpec((NQ, _RB, CH), lambda i: (0, i, 0)),
                  pl.BlockSpec((NQ, _RB, CH), lambda i: (0, i, 0)),
                  pl.BlockSpec((C, C), lambda i: (0, 0)),
                  pl.BlockSpec((1, 1), lambda i: (0, 0))],
        out_specs=pl.BlockSpec((NQ, _RB, CH), lambda i: (0, i, 0)),
        out_shape=jax.ShapeDtypeStruct((NQ, N_NODES, CH), jnp.float32),
    )(xp, xv, w, slope)


def _epi_body(xp_ref, xv_ref, o_ref):
    a = xp_ref[0] + xv_ref[0]
    o_ref[...] = jnp.where(a >= 0, a, 0.01 * a)


def _epi(xp, xv):
    return pl.pallas_call(
        _epi_body,
        grid=(N_NODES // _RB,),
        in_specs=[pl.BlockSpec((1, _RB, CH), lambda i: (0, i, 0)),
                  pl.BlockSpec((1, _RB, CH), lambda i: (0, i, 0))],
        out_specs=pl.BlockSpec((_RB, CH), lambda i: (i, 0)),
        out_shape=jax.ShapeDtypeStruct((N_NODES, CH), jnp.float32),
    )(xp, xv)


def kernel(X, V, E, H, W1, W2, W3):
    del H
    w1p = jnp.pad(W1, ((0, C - NFEAT), (0, 0)))
    w3p = jnp.pad(W3, ((0, 0), (0, C - C3)))
    ws = jnp.stack([w1p, W2, w3p])
    slopes = jnp.array([1.0, 0.01, 0.01], jnp.float32).reshape(3, 1, 1)
    zeros_blk = jnp.zeros((ZROWS, CH), jnp.float32)
    ones_blk = jnp.ones((K, L), jnp.float32)

    xp0 = jnp.concatenate(
        [X.reshape(N_NODES, 2, CH).transpose(1, 0, 2),
         jnp.zeros((2, N_NODES, CH), jnp.float32)])
    xv0 = jnp.zeros_like(xp0)

    def step(carry, xs):
        xp_prev, xv_prev = carry
        w, slope = xs
        xp = _mmf(xp_prev, xv_prev, w, slope)
        xv = _sc_sparse(xp, V, E, zeros_blk, ones_blk)
        return (xp, xv), None

    (xp3, xv3), _ = lax.scan(step, (xp0, xv0), (ws, slopes))
    out = _epi(xp3, xv3)
    return out[:, :C3]


# speedup vs baseline: 5.6638x; 1.1741x over previous
import jax
import jax.numpy as jnp
from jax import lax
from jax.experimental import pallas as pl
from jax.experimental.pallas import tpu as pltpu
from jax.experimental.pallas import tpu_sc as plsc

N_NODES = 10000
NNZ = 320000
N_HEDGES = 5000
NFEAT = 128
C = 256
NQ = 4
CH = C // NQ
C3 = 40

NC, NS, L = 2, 16, 16
NPASS = NQ // NC
NH_PAD = 5008
PAIRS_PER_SUB = NNZ // NS
K = 400
NCHUNK = PAIRS_PER_SUB // K
ZROWS = 32
EROWS = NH_PAD // NS
VROWS = N_NODES // NS

_mesh = plsc.VectorSubcoreMesh(core_axis_name="c", subcore_axis_name="s")
_sc_params = pltpu.CompilerParams(use_tc_tiling_on_sc=False)


def _zero_slices(zsrc, dst_sh, row0, nrows):
    done = 0
    while done < nrows:
        nb = min(ZROWS, nrows - done)
        pltpu.sync_copy(zsrc.at[pl.ds(0, nb)], dst_sh.at[pl.ds(row0 + done, nb)])
        done += nb


def _gs_pipeline(s, v_hbm, e_hbm, gather_src, scat, bufs):
    vidx0, eidx0, rows0, sem0, vidx1, eidx1, rows1, sem1 = bufs
    strip = s * PAIRS_PER_SUB

    def load_idx(base, vb, eb):
        pltpu.sync_copy(v_hbm.at[pl.ds(base, K)], vb)
        pltpu.sync_copy(e_hbm.at[pl.ds(base, K)], eb)

    load_idx(strip, vidx0, eidx0)
    pltpu.async_copy(gather_src(vidx0, eidx0), rows0, sem0)

    @pl.loop(0, NCHUNK // 2)
    def _(u):
        t1 = strip + (2 * u + 1) * K
        t2 = strip + jnp.minimum((2 * u + 2) * K, (NCHUNK - 1) * K)
        load_idx(t1, vidx1, eidx1)
        pltpu.async_copy(gather_src(vidx1, eidx1), rows1, sem1)
        pltpu.make_async_copy(gather_src(vidx0, eidx0), rows0, sem0).wait()
        scat(rows0, vidx0, eidx0)
        load_idx(t2, vidx0, eidx0)
        pltpu.async_copy(gather_src(vidx0, eidx0), rows0, sem0)
        pltpu.make_async_copy(gather_src(vidx1, eidx1), rows1, sem1).wait()
        scat(rows1, vidx1, eidx1)

    pltpu.make_async_copy(gather_src(vidx0, eidx0), rows0, sem0).wait()


def _sc_body(xp_hbm, v_hbm, e_hbm, z_hbm, o_hbm, xv_out,
             vidx0, eidx0, rows0, vidx1, eidx1, rows1, zblk, nbuf, sbuf, ones,
             cnt_sh, esum_sh, xv_sh, sem0, sem1):
    c = lax.axis_index("c")
    s = lax.axis_index("s")
    bufs = (vidx0, eidx0, rows0, sem0, vidx1, eidx1, rows1, sem1)

    pltpu.sync_copy(z_hbm, zblk)
    pltpu.sync_copy(o_hbm, ones)

    for cs in range(NPASS):
        q = c * NPASS + cs

        _zero_slices(zblk, esum_sh, s * EROWS, EROWS)
        _zero_slices(zblk, xv_sh, s * VROWS, VROWS)
        if cs == 0:
            _zero_slices(zblk.at[:, pl.ds(0, L)], cnt_sh, s * EROWS, EROWS)
        plsc.subcore_barrier()

        if cs == 0:
            def scat_a(rows, vb, eb):
                pltpu.sync_copy(rows, esum_sh.at[eb], add=True)
                pltpu.sync_copy(ones, cnt_sh.at[eb], add=True)
        else:
            def scat_a(rows, vb, eb):
                pltpu.sync_copy(rows, esum_sh.at[eb], add=True)
        _gs_pipeline(s, v_hbm, e_hbm,
                     lambda vb, eb: xp_hbm.at[q].at[vb], scat_a, bufs)

        plsc.subcore_barrier()

        row0 = s * EROWS
        done = 0
        while done < EROWS:
            nb = min(ZROWS, EROWS - done)
            off = row0 + done
            pltpu.sync_copy(esum_sh.at[pl.ds(off, nb)], nbuf.at[pl.ds(0, nb)])
            pltpu.sync_copy(cnt_sh.at[pl.ds(off, nb)], sbuf.at[pl.ds(0, nb)])

            @pl.loop(0, nb)
            def _(r):
                scale = 1.0 / jnp.maximum(sbuf[r, pl.ds(0, L)], 1.0)
                for j in range(CH // L):
                    nbuf[r, pl.ds(j * L, L)] = nbuf[r, pl.ds(j * L, L)] * scale

            pltpu.sync_copy(nbuf.at[pl.ds(0, nb)], esum_sh.at[pl.ds(off, nb)])
            done += nb
        plsc.subcore_barrier()

        def scat_c(rows, vb, eb):
            pltpu.sync_copy(rows, xv_sh.at[vb], add=True)

        _gs_pipeline(s, v_hbm, e_hbm,
                     lambda vb, eb: esum_sh.at[eb], scat_c, bufs)

        plsc.subcore_barrier()

        pltpu.sync_copy(xv_sh.at[pl.ds(s * VROWS, VROWS)],
                        xv_out.at[q].at[pl.ds(s * VROWS, VROWS)])


_sc_sparse = pl.kernel(
    _sc_body,
    out_type=jax.ShapeDtypeStruct((NQ, N_NODES, CH), jnp.float32),
    mesh=_mesh,
    scratch_types=[
        pltpu.VMEM((K,), jnp.int32),
        pltpu.VMEM((K,), jnp.int32),
        pltpu.VMEM((K, CH), jnp.float32),
        pltpu.VMEM((K,), jnp.int32),
        pltpu.VMEM((K,), jnp.int32),
        pltpu.VMEM((K, CH), jnp.float32),
        pltpu.VMEM((ZROWS, CH), jnp.float32),
        pltpu.VMEM((ZROWS, CH), jnp.float32),
        pltpu.VMEM((ZROWS, L), jnp.float32),
        pltpu.VMEM((K, L), jnp.float32),
        pltpu.VMEM_SHARED((NH_PAD, L), jnp.float32),
        pltpu.VMEM_SHARED((NH_PAD, CH), jnp.float32),
        pltpu.VMEM_SHARED((N_NODES, CH), jnp.float32),
        pltpu.SemaphoreType.DMA,
        pltpu.SemaphoreType.DMA,
    ],
    compiler_params=_sc_params,
)


_RB = 2000
_PREC = jax.lax.Precision.HIGHEST


def _mmf_body(xp_ref, xv_ref, w_ref, c_ref, o_ref):
    a = jnp.concatenate(
        [xp_ref[i] + xv_ref[i] for i in range(NQ)], axis=1)
    a = jnp.maximum(a, 0.0) + jnp.minimum(a, 0.0) * c_ref[...]
    r = jnp.dot(a, w_ref[...], preferred_element_type=jnp.float32,
                precision=_PREC)
    for i in range(NQ):
        o_ref[i] = r[:, i * CH:(i + 1) * CH]


def _mmf(xp, xv, w, slope):
    return pl.pallas_call(
        _mmf_body,
        grid=(N_NODES // _RB,),
        in_specs=[pl.BlockSpec((NQ, _RB, CH), lambda i: (0, i, 0)),
                  pl.BlockSpec((NQ, _RB, CH), lambda i: (0, i, 0)),
                  pl.BlockSpec((C, C), lambda i: (0, 0)),
                  pl.BlockSpec((1, 1), lambda i: (0, 0))],
        out_specs=pl.BlockSpec((NQ, _RB, CH), lambda i: (0, i, 0)),
        out_shape=jax.ShapeDtypeStruct((NQ, N_NODES, CH), jnp.float32),
    )(xp, xv, w, slope)


def _epi_body(xp_ref, xv_ref, o_ref):
    a = xp_ref[0] + xv_ref[0]
    o_ref[...] = jnp.where(a >= 0, a, 0.01 * a)


def _epi(xp, xv):
    return pl.pallas_call(
        _epi_body,
        grid=(N_NODES // _RB,),
        in_specs=[pl.BlockSpec((1, _RB, CH), lambda i: (0, i, 0)),
                  pl.BlockSpec((1, _RB, CH), lambda i: (0, i, 0))],
        out_specs=pl.BlockSpec((_RB, CH), lambda i: (i, 0)),
        out_shape=jax.ShapeDtypeStruct((N_NODES, CH), jnp.float32),
    )(xp, xv)


def kernel(X, V, E, H, W1, W2, W3):
    del H
    w1p = jnp.pad(W1, ((0, C - NFEAT), (0, 0)))
    w3p = jnp.pad(W3, ((0, 0), (0, C - C3)))
    ws = jnp.stack([w1p, W2, w3p])
    slopes = jnp.array([1.0, 0.01, 0.01], jnp.float32).reshape(3, 1, 1)
    zeros_blk = jnp.zeros((ZROWS, CH), jnp.float32)
    ones_blk = jnp.ones((K, L), jnp.float32)

    xp0 = jnp.concatenate(
        [X.reshape(N_NODES, 2, CH).transpose(1, 0, 2),
         jnp.zeros((2, N_NODES, CH), jnp.float32)])
    xv0 = jnp.zeros_like(xp0)

    def step(carry, xs):
        xp_prev, xv_prev = carry
        w, slope = xs
        xp = _mmf(xp_prev, xv_prev, w, slope)
        xv = _sc_sparse(xp, V, E, zeros_blk, ones_blk)
        return (xp, xv), None

    (xp3, xv3), _ = lax.scan(step, (xp0, xv0), (ws, slopes))
    out = _epi(xp3, xv3)
    return out[:, :C3]


# speedup vs baseline: 5.6916x; 1.0049x over previous
import jax
import jax.numpy as jnp
from jax import lax
from jax.experimental import pallas as pl
from jax.experimental.pallas import tpu as pltpu
from jax.experimental.pallas import tpu_sc as plsc

N_NODES = 10000
NNZ = 320000
N_HEDGES = 5000
NFEAT = 128
C = 256
NQ = 4
CH = C // NQ
C3 = 40

NC, NS, L = 2, 16, 16
NPASS = NQ // NC
NH_PAD = 5008
PAIRS_PER_SUB = NNZ // NS
K = 400
NCHUNK = PAIRS_PER_SUB // K
ZROWS = 32
EROWS = NH_PAD // NS
VROWS = N_NODES // NS

_mesh = plsc.VectorSubcoreMesh(core_axis_name="c", subcore_axis_name="s")
_sc_params = pltpu.CompilerParams(use_tc_tiling_on_sc=False)


def _zero_slices(zsrc, dst_sh, row0, nrows):
    done = 0
    while done < nrows:
        nb = min(ZROWS, nrows - done)
        pltpu.sync_copy(zsrc.at[pl.ds(0, nb)], dst_sh.at[pl.ds(row0 + done, nb)])
        done += nb


def _gs_pipeline(s, v_hbm, e_hbm, gather_src, scat, bufs):
    vidx0, eidx0, rows0, sem0, vidx1, eidx1, rows1, sem1 = bufs
    strip = s * PAIRS_PER_SUB

    def load_idx(base, vb, eb):
        pltpu.sync_copy(v_hbm.at[pl.ds(base, K)], vb)
        pltpu.sync_copy(e_hbm.at[pl.ds(base, K)], eb)

    load_idx(strip, vidx0, eidx0)
    pltpu.async_copy(gather_src(vidx0, eidx0), rows0, sem0)

    @pl.loop(0, NCHUNK // 2)
    def _(u):
        t1 = strip + (2 * u + 1) * K
        t2 = strip + jnp.minimum((2 * u + 2) * K, (NCHUNK - 1) * K)
        load_idx(t1, vidx1, eidx1)
        pltpu.async_copy(gather_src(vidx1, eidx1), rows1, sem1)
        pltpu.make_async_copy(gather_src(vidx0, eidx0), rows0, sem0).wait()
        scat(rows0, vidx0, eidx0)
        load_idx(t2, vidx0, eidx0)
        pltpu.async_copy(gather_src(vidx0, eidx0), rows0, sem0)
        pltpu.make_async_copy(gather_src(vidx1, eidx1), rows1, sem1).wait()
        scat(rows1, vidx1, eidx1)

    pltpu.make_async_copy(gather_src(vidx0, eidx0), rows0, sem0).wait()


def _sc_body(xp_hbm, v_hbm, e_hbm, z_hbm, o_hbm, xv_out,
             vidx0, eidx0, rows0, vidx1, eidx1, rows1, zblk, nbuf, sbuf, ones,
             cnt_sh, esum_sh, xv_sh, sem0, sem1):
    c = lax.axis_index("c")
    s = lax.axis_index("s")
    bufs = (vidx0, eidx0, rows0, sem0, vidx1, eidx1, rows1, sem1)

    pltpu.sync_copy(z_hbm, zblk)
    pltpu.sync_copy(o_hbm, ones)

    for cs in range(NPASS):
        q = c * NPASS + cs

        _zero_slices(zblk, esum_sh, s * EROWS, EROWS)
        _zero_slices(zblk, xv_sh, s * VROWS, VROWS)
        if cs == 0:
            _zero_slices(zblk.at[:, pl.ds(0, L)], cnt_sh, s * EROWS, EROWS)
        plsc.subcore_barrier()
        _ctx_a = jax.named_scope("phaseA"); _ctx_a.__enter__()

        if cs == 0:
            def scat_a(rows, vb, eb):
                pltpu.sync_copy(rows, esum_sh.at[eb], add=True)
                pltpu.sync_copy(ones, cnt_sh.at[eb], add=True)
        else:
            def scat_a(rows, vb, eb):
                pltpu.sync_copy(rows, esum_sh.at[eb], add=True)
        _gs_pipeline(s, v_hbm, e_hbm,
                     lambda vb, eb: xp_hbm.at[q].at[vb], scat_a, bufs)

        _ctx_a.__exit__(None, None, None)
        plsc.subcore_barrier()

        row0 = s * EROWS
        done = 0
        while done < EROWS:
            nb = min(ZROWS, EROWS - done)
            off = row0 + done
            pltpu.sync_copy(esum_sh.at[pl.ds(off, nb)], nbuf.at[pl.ds(0, nb)])
            pltpu.sync_copy(cnt_sh.at[pl.ds(off, nb)], sbuf.at[pl.ds(0, nb)])

            @pl.loop(0, nb)
            def _(r):
                scale = 1.0 / jnp.maximum(sbuf[r, pl.ds(0, L)], 1.0)
                for j in range(CH // L):
                    nbuf[r, pl.ds(j * L, L)] = nbuf[r, pl.ds(j * L, L)] * scale

            pltpu.sync_copy(nbuf.at[pl.ds(0, nb)], esum_sh.at[pl.ds(off, nb)])
            done += nb
        plsc.subcore_barrier()

        def scat_c(rows, vb, eb):
            pltpu.sync_copy(rows, xv_sh.at[vb], add=True)

        with jax.named_scope("phaseC"):
            _gs_pipeline(s, v_hbm, e_hbm,
                         lambda vb, eb: esum_sh.at[eb], scat_c, bufs)

        plsc.subcore_barrier()

        pltpu.sync_copy(xv_sh.at[pl.ds(s * VROWS, VROWS)],
                        xv_out.at[q].at[pl.ds(s * VROWS, VROWS)])


_sc_sparse = pl.kernel(
    _sc_body,
    out_type=jax.ShapeDtypeStruct((NQ, N_NODES, CH), jnp.float32),
    mesh=_mesh,
    scratch_types=[
        pltpu.VMEM((K,), jnp.int32),
        pltpu.VMEM((K,), jnp.int32),
        pltpu.VMEM((K, CH), jnp.float32),
        pltpu.VMEM((K,), jnp.int32),
        pltpu.VMEM((K,), jnp.int32),
        pltpu.VMEM((K, CH), jnp.float32),
        pltpu.VMEM((ZROWS, CH), jnp.float32),
        pltpu.VMEM((ZROWS, CH), jnp.float32),
        pltpu.VMEM((ZROWS, L), jnp.float32),
        pltpu.VMEM((K, L), jnp.float32),
        pltpu.VMEM_SHARED((NH_PAD, L), jnp.float32),
        pltpu.VMEM_SHARED((NH_PAD, CH), jnp.float32),
        pltpu.VMEM_SHARED((N_NODES, CH), jnp.float32),
        pltpu.SemaphoreType.DMA,
        pltpu.SemaphoreType.DMA,
    ],
    compiler_params=_sc_params,
)


_RB = 2000
_PREC = jax.lax.Precision.HIGHEST


def _mmf_body(xp_ref, xv_ref, w_ref, c_ref, o_ref):
    a = jnp.concatenate(
        [xp_ref[i] + xv_ref[i] for i in range(NQ)], axis=1)
    a = jnp.maximum(a, 0.0) + jnp.minimum(a, 0.0) * c_ref[...]
    r = jnp.dot(a, w_ref[...], preferred_element_type=jnp.float32,
                precision=_PREC)
    for i in range(NQ):
        o_ref[i] = r[:, i * CH:(i + 1) * CH]


def _mmf(xp, xv, w, slope):
    return pl.pallas_call(
        _mmf_body,
        grid=(N_NODES // _RB,),
        in_specs=[pl.BlockSpec((NQ, _RB, CH), lambda i: (0, i, 0)),
                  pl.BlockSpec((NQ, _RB, CH), lambda i: (0, i, 0)),
                  pl.BlockSpec((C, C), lambda i: (0, 0)),
                  pl.BlockSpec((1, 1), lambda i: (0, 0))],
        out_specs=pl.BlockSpec((NQ, _RB, CH), lambda i: (0, i, 0)),
        out_shape=jax.ShapeDtypeStruct((NQ, N_NODES, CH), jnp.float32),
    )(xp, xv, w, slope)


def _epi_body(xp_ref, xv_ref, o_ref):
    a = xp_ref[0] + xv_ref[0]
    o_ref[...] = jnp.where(a >= 0, a, 0.01 * a)


def _epi(xp, xv):
    return pl.pallas_call(
        _epi_body,
        grid=(N_NODES // _RB,),
        in_specs=[pl.BlockSpec((1, _RB, CH), lambda i: (0, i, 0)),
                  pl.BlockSpec((1, _RB, CH), lambda i: (0, i, 0))],
        out_specs=pl.BlockSpec((_RB, CH), lambda i: (i, 0)),
        out_shape=jax.ShapeDtypeStruct((N_NODES, CH), jnp.float32),
    )(xp, xv)


def kernel(X, V, E, H, W1, W2, W3):
    del H
    w1p = jnp.pad(W1, ((0, C - NFEAT), (0, 0)))
    w3p = jnp.pad(W3, ((0, 0), (0, C - C3)))
    ws = jnp.stack([w1p, W2, w3p])
    slopes = jnp.array([1.0, 0.01, 0.01], jnp.float32).reshape(3, 1, 1)
    zeros_blk = jnp.zeros((ZROWS, CH), jnp.float32)
    ones_blk = jnp.ones((K, L), jnp.float32)

    xp0 = jnp.concatenate(
        [X.reshape(N_NODES, 2, CH).transpose(1, 0, 2),
         jnp.zeros((2, N_NODES, CH), jnp.float32)])
    xv0 = jnp.zeros_like(xp0)

    def step(carry, xs):
        xp_prev, xv_prev = carry
        w, slope = xs
        xp = _mmf(xp_prev, xv_prev, w, slope)
        xv = _sc_sparse(xp, V, E, zeros_blk, ones_blk)
        return (xp, xv), None

    (xp3, xv3), _ = lax.scan(step, (xp0, xv0), (ws, slopes))
    out = _epi(xp3, xv3)
    return out[:, :C3]


# speedup vs baseline: 6.7195x; 1.1806x over previous
import jax
import jax.numpy as jnp
from jax import lax
from jax.experimental import pallas as pl
from jax.experimental.pallas import tpu as pltpu
from jax.experimental.pallas import tpu_sc as plsc

N_NODES = 10000
NNZ = 320000
N_HEDGES = 5000
NFEAT = 128
C = 256
NQ = 4
CH = C // NQ
C3 = 40

NC, NS, L = 2, 16, 16
NPASS = NQ // NC
NH_PAD = 5008
PAIRS_PER_SUB = NNZ // NS
K = 400
NCHUNK = PAIRS_PER_SUB // K
ZROWS = 32
EROWS = NH_PAD // NS
VROWS = N_NODES // NS

_mesh = plsc.VectorSubcoreMesh(core_axis_name="c", subcore_axis_name="s")
_sc_params = pltpu.CompilerParams(use_tc_tiling_on_sc=False)


def _zero_slices(zsrc, dst_sh, row0, nrows):
    done = 0
    while done < nrows:
        nb = min(ZROWS, nrows - done)
        pltpu.sync_copy(zsrc.at[pl.ds(0, nb)], dst_sh.at[pl.ds(row0 + done, nb)])
        done += nb


def _gs_pipeline(s, v_hbm, e_hbm, gather_src, scat, bufs):
    vidx0, eidx0, rows0, sem0, vidx1, eidx1, rows1, sem1 = bufs
    strip = s * PAIRS_PER_SUB

    def load_idx(base, vb, eb):
        pltpu.sync_copy(v_hbm.at[pl.ds(base, K)], vb)
        pltpu.sync_copy(e_hbm.at[pl.ds(base, K)], eb)

    load_idx(strip, vidx0, eidx0)
    pltpu.async_copy(gather_src(vidx0, eidx0), rows0, sem0)

    @pl.loop(0, NCHUNK // 2)
    def _(u):
        t1 = strip + (2 * u + 1) * K
        t2 = strip + jnp.minimum((2 * u + 2) * K, (NCHUNK - 1) * K)
        load_idx(t1, vidx1, eidx1)
        pltpu.async_copy(gather_src(vidx1, eidx1), rows1, sem1)
        pltpu.make_async_copy(gather_src(vidx0, eidx0), rows0, sem0).wait()
        scat(rows0, vidx0, eidx0)
        load_idx(t2, vidx0, eidx0)
        pltpu.async_copy(gather_src(vidx0, eidx0), rows0, sem0)
        pltpu.make_async_copy(gather_src(vidx1, eidx1), rows1, sem1).wait()
        scat(rows1, vidx1, eidx1)

    pltpu.make_async_copy(gather_src(vidx0, eidx0), rows0, sem0).wait()


def _sc_body(xp_hbm, v_hbm, e_hbm, z_hbm, o_hbm, xv_out, xe_out,
             vidx0, eidx0, rows0, vidx1, eidx1, rows1, zblk, nbuf, sbuf, ones,
             cnt_sh, esum_sh, xv_sh, sem0, sem1):
    c = lax.axis_index("c")
    s = lax.axis_index("s")
    bufs = (vidx0, eidx0, rows0, sem0, vidx1, eidx1, rows1, sem1)

    pltpu.sync_copy(z_hbm, zblk)
    pltpu.sync_copy(o_hbm, ones)

    for cs in range(NPASS):
        q = c * NPASS + cs

        _zero_slices(zblk, esum_sh, s * EROWS, EROWS)
        _zero_slices(zblk, xv_sh, s * VROWS, VROWS)
        if cs == 0:
            _zero_slices(zblk.at[:, pl.ds(0, L)], cnt_sh, s * EROWS, EROWS)
        plsc.subcore_barrier()
        _ctx_a = jax.named_scope("phaseA"); _ctx_a.__enter__()

        if cs == 0:
            def scat_a(rows, vb, eb):
                pltpu.sync_copy(rows, esum_sh.at[eb], add=True)
                pltpu.sync_copy(ones, cnt_sh.at[eb], add=True)
        else:
            def scat_a(rows, vb, eb):
                pltpu.sync_copy(rows, esum_sh.at[eb], add=True)
        _gs_pipeline(s, v_hbm, e_hbm,
                     lambda vb, eb: xp_hbm.at[q].at[vb], scat_a, bufs)

        _ctx_a.__exit__(None, None, None)
        plsc.subcore_barrier()

        row0 = s * EROWS
        done = 0
        while done < EROWS:
            nb = min(ZROWS, EROWS - done)
            off = row0 + done
            pltpu.sync_copy(esum_sh.at[pl.ds(off, nb)], nbuf.at[pl.ds(0, nb)])
            pltpu.sync_copy(cnt_sh.at[pl.ds(off, nb)], sbuf.at[pl.ds(0, nb)])

            @pl.loop(0, nb)
            def _(r):
                scale = 1.0 / jnp.maximum(sbuf[r, pl.ds(0, L)], 1.0)
                for j in range(CH // L):
                    nbuf[r, pl.ds(j * L, L)] = nbuf[r, pl.ds(j * L, L)] * scale

            pltpu.sync_copy(nbuf.at[pl.ds(0, nb)], xe_out.at[q].at[pl.ds(off, nb)])
            done += nb
        plsc.subcore_barrier()

        def scat_c(rows, vb, eb):
            pltpu.sync_copy(rows, xv_sh.at[vb], add=True)

        with jax.named_scope("phaseC"):
            _gs_pipeline(s, v_hbm, e_hbm,
                         lambda vb, eb: xe_out.at[q].at[eb], scat_c, bufs)

        plsc.subcore_barrier()

        pltpu.sync_copy(xv_sh.at[pl.ds(s * VROWS, VROWS)],
                        xv_out.at[q].at[pl.ds(s * VROWS, VROWS)])


_sc_sparse = pl.kernel(
    _sc_body,
    out_type=[jax.ShapeDtypeStruct((NQ, N_NODES, CH), jnp.float32),
              jax.ShapeDtypeStruct((NQ, NH_PAD, CH), jnp.float32)],
    mesh=_mesh,
    scratch_types=[
        pltpu.VMEM((K,), jnp.int32),
        pltpu.VMEM((K,), jnp.int32),
        pltpu.VMEM((K, CH), jnp.float32),
        pltpu.VMEM((K,), jnp.int32),
        pltpu.VMEM((K,), jnp.int32),
        pltpu.VMEM((K, CH), jnp.float32),
        pltpu.VMEM((ZROWS, CH), jnp.float32),
        pltpu.VMEM((ZROWS, CH), jnp.float32),
        pltpu.VMEM((ZROWS, L), jnp.float32),
        pltpu.VMEM((K, L), jnp.float32),
        pltpu.VMEM_SHARED((NH_PAD, L), jnp.float32),
        pltpu.VMEM_SHARED((NH_PAD, CH), jnp.float32),
        pltpu.VMEM_SHARED((N_NODES, CH), jnp.float32),
        pltpu.SemaphoreType.DMA,
        pltpu.SemaphoreType.DMA,
    ],
    compiler_params=_sc_params,
)


_RB = 2000
_PREC = jax.lax.Precision.HIGHEST


def _mmf_body(xp_ref, xv_ref, w_ref, c_ref, o_ref):
    a = jnp.concatenate(
        [xp_ref[i] + xv_ref[i] for i in range(NQ)], axis=1)
    a = jnp.maximum(a, 0.0) + jnp.minimum(a, 0.0) * c_ref[...]
    r = jnp.dot(a, w_ref[...], preferred_element_type=jnp.float32,
                precision=_PREC)
    for i in range(NQ):
        o_ref[i] = r[:, i * CH:(i + 1) * CH]


def _mmf(xp, xv, w, slope):
    return pl.pallas_call(
        _mmf_body,
        grid=(N_NODES // _RB,),
        in_specs=[pl.BlockSpec((NQ, _RB, CH), lambda i: (0, i, 0)),
                  pl.BlockSpec((NQ, _RB, CH), lambda i: (0, i, 0)),
                  pl.BlockSpec((C, C), lambda i: (0, 0)),
                  pl.BlockSpec((1, 1), lambda i: (0, 0))],
        out_specs=pl.BlockSpec((NQ, _RB, CH), lambda i: (0, i, 0)),
        out_shape=jax.ShapeDtypeStruct((NQ, N_NODES, CH), jnp.float32),
    )(xp, xv, w, slope)


def _epi_body(xp_ref, xv_ref, o_ref):
    a = xp_ref[0] + xv_ref[0]
    o_ref[...] = jnp.where(a >= 0, a, 0.01 * a)


def _epi(xp, xv):
    return pl.pallas_call(
        _epi_body,
        grid=(N_NODES // _RB,),
        in_specs=[pl.BlockSpec((1, _RB, CH), lambda i: (0, i, 0)),
                  pl.BlockSpec((1, _RB, CH), lambda i: (0, i, 0))],
        out_specs=pl.BlockSpec((_RB, CH), lambda i: (i, 0)),
        out_shape=jax.ShapeDtypeStruct((N_NODES, CH), jnp.float32),
    )(xp, xv)


def kernel(X, V, E, H, W1, W2, W3):
    del H
    w1p = jnp.pad(W1, ((0, C - NFEAT), (0, 0)))
    w3p = jnp.pad(W3, ((0, 0), (0, C - C3)))
    ws = jnp.stack([w1p, W2, w3p])
    slopes = jnp.array([1.0, 0.01, 0.01], jnp.float32).reshape(3, 1, 1)
    zeros_blk = jnp.zeros((ZROWS, CH), jnp.float32)
    ones_blk = jnp.ones((K, L), jnp.float32)

    xp0 = jnp.concatenate(
        [X.reshape(N_NODES, 2, CH).transpose(1, 0, 2),
         jnp.zeros((2, N_NODES, CH), jnp.float32)])
    xv0 = jnp.zeros_like(xp0)

    def step(carry, xs):
        xp_prev, xv_prev = carry
        w, slope = xs
        xp = _mmf(xp_prev, xv_prev, w, slope)
        xv, _ = _sc_sparse(xp, V, E, zeros_blk, ones_blk)
        return (xp, xv), None

    (xp3, xv3), _ = lax.scan(step, (xp0, xv0), (ws, slopes))
    out = _epi(xp3, xv3)
    return out[:, :C3]


# speedup vs baseline: 6.9156x; 1.0292x over previous
import jax
import jax.numpy as jnp
from jax import lax
from jax.experimental import pallas as pl
from jax.experimental.pallas import tpu as pltpu
from jax.experimental.pallas import tpu_sc as plsc

N_NODES = 10000
NNZ = 320000
N_HEDGES = 5000
NFEAT = 128
C = 256
NQ = 4
CH = C // NQ
C3 = 40

NC, NS, L = 2, 16, 16
NPASS = NQ // NC
NH_PAD = 5008
PAIRS_PER_SUB = NNZ // NS
K = 400
NCHUNK = PAIRS_PER_SUB // K
ZROWS = 32
EROWS = NH_PAD // NS
VROWS = N_NODES // NS

_mesh = plsc.VectorSubcoreMesh(core_axis_name="c", subcore_axis_name="s")
_sc_params = pltpu.CompilerParams(use_tc_tiling_on_sc=False)


def _zero_slices(zsrc, dst_sh, row0, nrows):
    done = 0
    while done < nrows:
        nb = min(ZROWS, nrows - done)
        pltpu.sync_copy(zsrc.at[pl.ds(0, nb)], dst_sh.at[pl.ds(row0 + done, nb)])
        done += nb


def _gs_pipeline(s, v_hbm, e_hbm, gather_src, scat, scat_wait, bufs):
    (vidx0, eidx0, rows0, sem0, ssem0,
     vidx1, eidx1, rows1, sem1, ssem1) = bufs
    strip = s * PAIRS_PER_SUB

    def load_idx(base, vb, eb):
        pltpu.sync_copy(v_hbm.at[pl.ds(base, K)], vb)
        pltpu.sync_copy(e_hbm.at[pl.ds(base, K)], eb)

    def gather_go(base, vb, eb, rows, gsem):
        load_idx(base, vb, eb)
        pltpu.async_copy(gather_src(vb, eb), rows, gsem)

    def gather_wait(vb, eb, rows, gsem):
        pltpu.make_async_copy(gather_src(vb, eb), rows, gsem).wait()

    gather_go(strip, vidx0, eidx0, rows0, sem0)
    gather_go(strip + K, vidx1, eidx1, rows1, sem1)
    gather_wait(vidx0, eidx0, rows0, sem0)
    scat(rows0, vidx0, eidx0, ssem0)
    gather_wait(vidx1, eidx1, rows1, sem1)
    scat(rows1, vidx1, eidx1, ssem1)

    @pl.loop(1, NCHUNK // 2)
    def _(u):
        a = strip + (2 * u) * K
        scat_wait(rows0, vidx0, eidx0, ssem0)
        gather_go(a, vidx0, eidx0, rows0, sem0)
        scat_wait(rows1, vidx1, eidx1, ssem1)
        gather_go(a + K, vidx1, eidx1, rows1, sem1)
        gather_wait(vidx0, eidx0, rows0, sem0)
        scat(rows0, vidx0, eidx0, ssem0)
        gather_wait(vidx1, eidx1, rows1, sem1)
        scat(rows1, vidx1, eidx1, ssem1)

    scat_wait(rows0, vidx0, eidx0, ssem0)
    scat_wait(rows1, vidx1, eidx1, ssem1)


def _sc_body(xp_hbm, v_hbm, e_hbm, z_hbm, o_hbm, xv_out, xe_out,
             vidx0, eidx0, rows0, vidx1, eidx1, rows1, zblk, nbuf, sbuf, ones,
             cnt_sh, esum_sh, xv_sh, sem0, sem1, ssem0, ssem1):
    c = lax.axis_index("c")
    s = lax.axis_index("s")
    bufs = (vidx0, eidx0, rows0, sem0, ssem0,
            vidx1, eidx1, rows1, sem1, ssem1)

    pltpu.sync_copy(z_hbm, zblk)
    pltpu.sync_copy(o_hbm, ones)

    for cs in range(NPASS):
        q = c * NPASS + cs

        _zero_slices(zblk, esum_sh, s * EROWS, EROWS)
        _zero_slices(zblk, xv_sh, s * VROWS, VROWS)
        if cs == 0:
            _zero_slices(zblk.at[:, pl.ds(0, L)], cnt_sh, s * EROWS, EROWS)
        plsc.subcore_barrier()
        _ctx_a = jax.named_scope("phaseA"); _ctx_a.__enter__()

        if cs == 0:
            def scat_a(rows, vb, eb, ssem):
                pltpu.async_copy(rows, esum_sh.at[eb], ssem, add=True)
                pltpu.async_copy(ones, cnt_sh.at[eb], ssem, add=True)

            def scat_a_wait(rows, vb, eb, ssem):
                pltpu.make_async_copy(rows, esum_sh.at[eb], ssem).wait()
                pltpu.make_async_copy(ones, cnt_sh.at[eb], ssem).wait()
        else:
            def scat_a(rows, vb, eb, ssem):
                pltpu.async_copy(rows, esum_sh.at[eb], ssem, add=True)

            def scat_a_wait(rows, vb, eb, ssem):
                pltpu.make_async_copy(rows, esum_sh.at[eb], ssem).wait()
        _gs_pipeline(s, v_hbm, e_hbm,
                     lambda vb, eb: xp_hbm.at[q].at[vb], scat_a, scat_a_wait,
                     bufs)

        _ctx_a.__exit__(None, None, None)
        plsc.subcore_barrier()

        row0 = s * EROWS
        done = 0
        while done < EROWS:
            nb = min(ZROWS, EROWS - done)
            off = row0 + done
            pltpu.sync_copy(esum_sh.at[pl.ds(off, nb)], nbuf.at[pl.ds(0, nb)])
            pltpu.sync_copy(cnt_sh.at[pl.ds(off, nb)], sbuf.at[pl.ds(0, nb)])

            @pl.loop(0, nb)
            def _(r):
                scale = 1.0 / jnp.maximum(sbuf[r, pl.ds(0, L)], 1.0)
                for j in range(CH // L):
                    nbuf[r, pl.ds(j * L, L)] = nbuf[r, pl.ds(j * L, L)] * scale

            pltpu.sync_copy(nbuf.at[pl.ds(0, nb)], xe_out.at[q].at[pl.ds(off, nb)])
            done += nb
        plsc.subcore_barrier()

        def scat_c(rows, vb, eb, ssem):
            pltpu.async_copy(rows, xv_sh.at[vb], ssem, add=True)

        def scat_c_wait(rows, vb, eb, ssem):
            pltpu.make_async_copy(rows, xv_sh.at[vb], ssem).wait()

        with jax.named_scope("phaseC"):
            _gs_pipeline(s, v_hbm, e_hbm,
                         lambda vb, eb: xe_out.at[q].at[eb], scat_c,
                         scat_c_wait, bufs)

        plsc.subcore_barrier()

        pltpu.sync_copy(xv_sh.at[pl.ds(s * VROWS, VROWS)],
                        xv_out.at[q].at[pl.ds(s * VROWS, VROWS)])


_sc_sparse = pl.kernel(
    _sc_body,
    out_type=[jax.ShapeDtypeStruct((NQ, N_NODES, CH), jnp.float32),
              jax.ShapeDtypeStruct((NQ, NH_PAD, CH), jnp.float32)],
    mesh=_mesh,
    scratch_types=[
        pltpu.VMEM((K,), jnp.int32),
        pltpu.VMEM((K,), jnp.int32),
        pltpu.VMEM((K, CH), jnp.float32),
        pltpu.VMEM((K,), jnp.int32),
        pltpu.VMEM((K,), jnp.int32),
        pltpu.VMEM((K, CH), jnp.float32),
        pltpu.VMEM((ZROWS, CH), jnp.float32),
        pltpu.VMEM((ZROWS, CH), jnp.float32),
        pltpu.VMEM((ZROWS, L), jnp.float32),
        pltpu.VMEM((K, L), jnp.float32),
        pltpu.VMEM_SHARED((NH_PAD, L), jnp.float32),
        pltpu.VMEM_SHARED((NH_PAD, CH), jnp.float32),
        pltpu.VMEM_SHARED((N_NODES, CH), jnp.float32),
        pltpu.SemaphoreType.DMA,
        pltpu.SemaphoreType.DMA,
        pltpu.SemaphoreType.DMA,
        pltpu.SemaphoreType.DMA,
    ],
    compiler_params=_sc_params,
)


_RB = 2000
_PREC = jax.lax.Precision.HIGHEST


def _mmf_body(xp_ref, xv_ref, w_ref, c_ref, o_ref):
    a = jnp.concatenate(
        [xp_ref[i] + xv_ref[i] for i in range(NQ)], axis=1)
    a = jnp.maximum(a, 0.0) + jnp.minimum(a, 0.0) * c_ref[...]
    r = jnp.dot(a, w_ref[...], preferred_element_type=jnp.float32,
                precision=_PREC)
    for i in range(NQ):
        o_ref[i] = r[:, i * CH:(i + 1) * CH]


def _mmf(xp, xv, w, slope):
    return pl.pallas_call(
        _mmf_body,
        grid=(N_NODES // _RB,),
        in_specs=[pl.BlockSpec((NQ, _RB, CH), lambda i: (0, i, 0)),
                  pl.BlockSpec((NQ, _RB, CH), lambda i: (0, i, 0)),
                  pl.BlockSpec((C, C), lambda i: (0, 0)),
                  pl.BlockSpec((1, 1), lambda i: (0, 0))],
        out_specs=pl.BlockSpec((NQ, _RB, CH), lambda i: (0, i, 0)),
        out_shape=jax.ShapeDtypeStruct((NQ, N_NODES, CH), jnp.float32),
    )(xp, xv, w, slope)


def _epi_body(xp_ref, xv_ref, o_ref):
    a = xp_ref[0] + xv_ref[0]
    o_ref[...] = jnp.where(a >= 0, a, 0.01 * a)


def _epi(xp, xv):
    return pl.pallas_call(
        _epi_body,
        grid=(N_NODES // _RB,),
        in_specs=[pl.BlockSpec((1, _RB, CH), lambda i: (0, i, 0)),
                  pl.BlockSpec((1, _RB, CH), lambda i: (0, i, 0))],
        out_specs=pl.BlockSpec((_RB, CH), lambda i: (i, 0)),
        out_shape=jax.ShapeDtypeStruct((N_NODES, CH), jnp.float32),
    )(xp, xv)


def kernel(X, V, E, H, W1, W2, W3):
    del H
    w1p = jnp.pad(W1, ((0, C - NFEAT), (0, 0)))
    w3p = jnp.pad(W3, ((0, 0), (0, C - C3)))
    ws = jnp.stack([w1p, W2, w3p])
    slopes = jnp.array([1.0, 0.01, 0.01], jnp.float32).reshape(3, 1, 1)
    zeros_blk = jnp.zeros((ZROWS, CH), jnp.float32)
    ones_blk = jnp.ones((K, L), jnp.float32)

    xp0 = jnp.concatenate(
        [X.reshape(N_NODES, 2, CH).transpose(1, 0, 2),
         jnp.zeros((2, N_NODES, CH), jnp.float32)])
    xv0 = jnp.zeros_like(xp0)

    def step(carry, xs):
        xp_prev, xv_prev = carry
        w, slope = xs
        xp = _mmf(xp_prev, xv_prev, w, slope)
        xv, _ = _sc_sparse(xp, V, E, zeros_blk, ones_blk)
        return (xp, xv), None

    (xp3, xv3), _ = lax.scan(step, (xp0, xv0), (ws, slopes))
    out = _epi(xp3, xv3)
    return out[:, :C3]


# speedup vs baseline: 7.0262x; 1.0160x over previous
import jax
import jax.numpy as jnp
from jax import lax
from jax.experimental import pallas as pl
from jax.experimental.pallas import tpu as pltpu
from jax.experimental.pallas import tpu_sc as plsc

N_NODES = 10000
NNZ = 320000
N_HEDGES = 5000
NFEAT = 128
C = 256
NQ = 4
CH = C // NQ
C3 = 40

NC, NS, L = 2, 16, 16
NPASS = NQ // NC
NH_PAD = 5008
PAIRS_PER_SUB = NNZ // NS
K = 400
NCHUNK = PAIRS_PER_SUB // K
ZROWS = 32
EROWS = NH_PAD // NS
VROWS = N_NODES // NS

_mesh = plsc.VectorSubcoreMesh(core_axis_name="c", subcore_axis_name="s")
_sc_params = pltpu.CompilerParams(use_tc_tiling_on_sc=False)


def _zero_issue(zsrc, dst_sh, row0, nrows, sem):
    done = 0
    while done < nrows:
        nb = min(ZROWS, nrows - done)
        pltpu.async_copy(zsrc.at[pl.ds(0, nb)], dst_sh.at[pl.ds(row0 + done, nb)], sem)
        done += nb


def _zero_wait(zsrc, dst_sh, row0, nrows, sem):
    done = 0
    while done < nrows:
        nb = min(ZROWS, nrows - done)
        pltpu.make_async_copy(zsrc.at[pl.ds(0, nb)],
                              dst_sh.at[pl.ds(row0 + done, nb)], sem).wait()
        done += nb


def _gs_pipeline(s, v_hbm, e_hbm, gather_src, scat, scat_wait, bufs):
    (vidx0, eidx0, rows0, sem0, ssem0,
     vidx1, eidx1, rows1, sem1, ssem1) = bufs
    strip = s * PAIRS_PER_SUB

    def load_idx(base, vb, eb):
        pltpu.sync_copy(v_hbm.at[pl.ds(base, K)], vb)
        pltpu.sync_copy(e_hbm.at[pl.ds(base, K)], eb)

    def gather_go(base, vb, eb, rows, gsem):
        load_idx(base, vb, eb)
        pltpu.async_copy(gather_src(vb, eb), rows, gsem)

    def gather_wait(vb, eb, rows, gsem):
        pltpu.make_async_copy(gather_src(vb, eb), rows, gsem).wait()

    gather_go(strip, vidx0, eidx0, rows0, sem0)
    gather_go(strip + K, vidx1, eidx1, rows1, sem1)
    gather_wait(vidx0, eidx0, rows0, sem0)
    scat(rows0, vidx0, eidx0, ssem0)
    gather_wait(vidx1, eidx1, rows1, sem1)
    scat(rows1, vidx1, eidx1, ssem1)

    @pl.loop(1, NCHUNK // 2)
    def _(u):
        a = strip + (2 * u) * K
        scat_wait(rows0, vidx0, eidx0, ssem0)
        gather_go(a, vidx0, eidx0, rows0, sem0)
        scat_wait(rows1, vidx1, eidx1, ssem1)
        gather_go(a + K, vidx1, eidx1, rows1, sem1)
        gather_wait(vidx0, eidx0, rows0, sem0)
        scat(rows0, vidx0, eidx0, ssem0)
        gather_wait(vidx1, eidx1, rows1, sem1)
        scat(rows1, vidx1, eidx1, ssem1)

    scat_wait(rows0, vidx0, eidx0, ssem0)
    scat_wait(rows1, vidx1, eidx1, ssem1)


def _sc_body(xp_hbm, v_hbm, e_hbm, z_hbm, o_hbm, xv_out, xe_out,
             vidx0, eidx0, rows0, vidx1, eidx1, rows1, zblk, nbuf, sbuf, ones,
             cnt_sh, esum_sh, xv_sh, sem0, sem1, ssem0, ssem1, zsem):
    c = lax.axis_index("c")
    s = lax.axis_index("s")
    bufs = (vidx0, eidx0, rows0, sem0, ssem0,
            vidx1, eidx1, rows1, sem1, ssem1)

    pltpu.sync_copy(z_hbm, zblk)
    pltpu.sync_copy(o_hbm, ones)
    _zero_issue(zblk, esum_sh, s * EROWS, EROWS, zsem)
    _zero_issue(zblk.at[:, pl.ds(0, L)], cnt_sh, s * EROWS, EROWS, zsem)
    _zero_wait(zblk, esum_sh, s * EROWS, EROWS, zsem)
    _zero_wait(zblk.at[:, pl.ds(0, L)], cnt_sh, s * EROWS, EROWS, zsem)
    plsc.subcore_barrier()

    for cs in range(NPASS):
        q = c * NPASS + cs

        _zero_issue(zblk, xv_sh, s * VROWS, VROWS, zsem)
        _ctx_a = jax.named_scope("phaseA"); _ctx_a.__enter__()

        if cs == 0:
            def scat_a(rows, vb, eb, ssem):
                pltpu.async_copy(rows, esum_sh.at[eb], ssem, add=True)
                pltpu.async_copy(ones, cnt_sh.at[eb], ssem, add=True)

            def scat_a_wait(rows, vb, eb, ssem):
                pltpu.make_async_copy(rows, esum_sh.at[eb], ssem).wait()
                pltpu.make_async_copy(ones, cnt_sh.at[eb], ssem).wait()
        else:
            def scat_a(rows, vb, eb, ssem):
                pltpu.async_copy(rows, esum_sh.at[eb], ssem, add=True)

            def scat_a_wait(rows, vb, eb, ssem):
                pltpu.make_async_copy(rows, esum_sh.at[eb], ssem).wait()
        _gs_pipeline(s, v_hbm, e_hbm,
                     lambda vb, eb: xp_hbm.at[q].at[vb], scat_a, scat_a_wait,
                     bufs)

        _ctx_a.__exit__(None, None, None)
        _zero_wait(zblk, xv_sh, s * VROWS, VROWS, zsem)
        plsc.subcore_barrier()

        row0 = s * EROWS
        done = 0
        while done < EROWS:
            nb = min(ZROWS, EROWS - done)
            off = row0 + done
            pltpu.sync_copy(esum_sh.at[pl.ds(off, nb)], nbuf.at[pl.ds(0, nb)])
            pltpu.sync_copy(cnt_sh.at[pl.ds(off, nb)], sbuf.at[pl.ds(0, nb)])

            @pl.loop(0, nb)
            def _(r):
                scale = 1.0 / jnp.maximum(sbuf[r, pl.ds(0, L)], 1.0)
                for j in range(CH // L):
                    nbuf[r, pl.ds(j * L, L)] = nbuf[r, pl.ds(j * L, L)] * scale

            pltpu.sync_copy(nbuf.at[pl.ds(0, nb)], xe_out.at[q].at[pl.ds(off, nb)])
            done += nb
        if cs + 1 < NPASS:
            _zero_issue(zblk, esum_sh, s * EROWS, EROWS, zsem)
        plsc.subcore_barrier()

        def scat_c(rows, vb, eb, ssem):
            pltpu.async_copy(rows, xv_sh.at[vb], ssem, add=True)

        def scat_c_wait(rows, vb, eb, ssem):
            pltpu.make_async_copy(rows, xv_sh.at[vb], ssem).wait()

        with jax.named_scope("phaseC"):
            _gs_pipeline(s, v_hbm, e_hbm,
                         lambda vb, eb: xe_out.at[q].at[eb], scat_c,
                         scat_c_wait, bufs)

        if cs + 1 < NPASS:
            _zero_wait(zblk, esum_sh, s * EROWS, EROWS, zsem)
        plsc.subcore_barrier()

        pltpu.sync_copy(xv_sh.at[pl.ds(s * VROWS, VROWS)],
                        xv_out.at[q].at[pl.ds(s * VROWS, VROWS)])


_sc_sparse = pl.kernel(
    _sc_body,
    out_type=[jax.ShapeDtypeStruct((NQ, N_NODES, CH), jnp.float32),
              jax.ShapeDtypeStruct((NQ, NH_PAD, CH), jnp.float32)],
    mesh=_mesh,
    scratch_types=[
        pltpu.VMEM((K,), jnp.int32),
        pltpu.VMEM((K,), jnp.int32),
        pltpu.VMEM((K, CH), jnp.float32),
        pltpu.VMEM((K,), jnp.int32),
        pltpu.VMEM((K,), jnp.int32),
        pltpu.VMEM((K, CH), jnp.float32),
        pltpu.VMEM((ZROWS, CH), jnp.float32),
        pltpu.VMEM((ZROWS, CH), jnp.float32),
        pltpu.VMEM((ZROWS, L), jnp.float32),
        pltpu.VMEM((K, L), jnp.float32),
        pltpu.VMEM_SHARED((NH_PAD, L), jnp.float32),
        pltpu.VMEM_SHARED((NH_PAD, CH), jnp.float32),
        pltpu.VMEM_SHARED((N_NODES, CH), jnp.float32),
        pltpu.SemaphoreType.DMA,
        pltpu.SemaphoreType.DMA,
        pltpu.SemaphoreType.DMA,
        pltpu.SemaphoreType.DMA,
        pltpu.SemaphoreType.DMA,
    ],
    compiler_params=_sc_params,
)


_RB = 2000
_PREC = jax.lax.Precision.HIGHEST


def _mmf_body(xp_ref, xv_ref, w_ref, c_ref, o_ref):
    a = jnp.concatenate(
        [xp_ref[i] + xv_ref[i] for i in range(NQ)], axis=1)
    a = jnp.maximum(a, 0.0) + jnp.minimum(a, 0.0) * c_ref[...]
    r = jnp.dot(a, w_ref[...], preferred_element_type=jnp.float32,
                precision=_PREC)
    for i in range(NQ):
        o_ref[i] = r[:, i * CH:(i + 1) * CH]


def _mmf(xp, xv, w, slope):
    return pl.pallas_call(
        _mmf_body,
        grid=(N_NODES // _RB,),
        in_specs=[pl.BlockSpec((NQ, _RB, CH), lambda i: (0, i, 0)),
                  pl.BlockSpec((NQ, _RB, CH), lambda i: (0, i, 0)),
                  pl.BlockSpec((C, C), lambda i: (0, 0)),
                  pl.BlockSpec((1, 1), lambda i: (0, 0))],
        out_specs=pl.BlockSpec((NQ, _RB, CH), lambda i: (0, i, 0)),
        out_shape=jax.ShapeDtypeStruct((NQ, N_NODES, CH), jnp.float32),
    )(xp, xv, w, slope)


def _epi_body(xp_ref, xv_ref, o_ref):
    a = xp_ref[0] + xv_ref[0]
    o_ref[...] = jnp.where(a >= 0, a, 0.01 * a)


def _epi(xp, xv):
    return pl.pallas_call(
        _epi_body,
        grid=(N_NODES // _RB,),
        in_specs=[pl.BlockSpec((1, _RB, CH), lambda i: (0, i, 0)),
                  pl.BlockSpec((1, _RB, CH), lambda i: (0, i, 0))],
        out_specs=pl.BlockSpec((_RB, CH), lambda i: (i, 0)),
        out_shape=jax.ShapeDtypeStruct((N_NODES, CH), jnp.float32),
    )(xp, xv)


def kernel(X, V, E, H, W1, W2, W3):
    del H
    w1p = jnp.pad(W1, ((0, C - NFEAT), (0, 0)))
    w3p = jnp.pad(W3, ((0, 0), (0, C - C3)))
    ws = jnp.stack([w1p, W2, w3p])
    slopes = jnp.array([1.0, 0.01, 0.01], jnp.float32).reshape(3, 1, 1)
    zeros_blk = jnp.zeros((ZROWS, CH), jnp.float32)
    ones_blk = jnp.ones((K, L), jnp.float32)

    xp0 = jnp.concatenate(
        [X.reshape(N_NODES, 2, CH).transpose(1, 0, 2),
         jnp.zeros((2, N_NODES, CH), jnp.float32)])
    xv0 = jnp.zeros_like(xp0)

    def step(carry, xs):
        xp_prev, xv_prev = carry
        w, slope = xs
        xp = _mmf(xp_prev, xv_prev, w, slope)
        xv, _ = _sc_sparse(xp, V, E, zeros_blk, ones_blk)
        return (xp, xv), None

    (xp3, xv3), _ = lax.scan(step, (xp0, xv0), (ws, slopes))
    out = _epi(xp3, xv3)
    return out[:, :C3]


# speedup vs baseline: 7.0662x; 1.0057x over previous
import jax
import jax.numpy as jnp
from jax import lax
from jax.experimental import pallas as pl
from jax.experimental.pallas import tpu as pltpu
from jax.experimental.pallas import tpu_sc as plsc

N_NODES = 10000
NNZ = 320000
N_HEDGES = 5000
NFEAT = 128
C = 256
NQ = 4
CH = C // NQ
C3 = 40

NC, NS, L = 2, 16, 16
NPASS = NQ // NC
NH_PAD = 5008
PAIRS_PER_SUB = NNZ // NS
K = 400
NCHUNK = PAIRS_PER_SUB // K
ZROWS = 32
EROWS = NH_PAD // NS
VROWS = N_NODES // NS

_mesh = plsc.VectorSubcoreMesh(core_axis_name="c", subcore_axis_name="s")
_sc_params = pltpu.CompilerParams(use_tc_tiling_on_sc=False)


def _zero_issue(zsrc, dst_sh, row0, nrows, sem):
    done = 0
    while done < nrows:
        nb = min(ZROWS, nrows - done)
        pltpu.async_copy(zsrc.at[pl.ds(0, nb)], dst_sh.at[pl.ds(row0 + done, nb)], sem)
        done += nb


def _zero_wait(zsrc, dst_sh, row0, nrows, sem):
    done = 0
    while done < nrows:
        nb = min(ZROWS, nrows - done)
        pltpu.make_async_copy(zsrc.at[pl.ds(0, nb)],
                              dst_sh.at[pl.ds(row0 + done, nb)], sem).wait()
        done += nb


def _gs_pipeline(s, v_hbm, e_hbm, gather_src, scat, scat_wait, bufs):
    (vidx0, eidx0, rows0, sem0, ssem0,
     vidx1, eidx1, rows1, sem1, ssem1) = bufs
    strip = s * PAIRS_PER_SUB

    def load_idx(base, vb, eb):
        pltpu.sync_copy(v_hbm.at[pl.ds(base, K)], vb)
        pltpu.sync_copy(e_hbm.at[pl.ds(base, K)], eb)

    def gather_go(base, vb, eb, rows, gsem):
        load_idx(base, vb, eb)
        pltpu.async_copy(gather_src(vb, eb), rows, gsem)

    def gather_wait(vb, eb, rows, gsem):
        pltpu.make_async_copy(gather_src(vb, eb), rows, gsem).wait()

    gather_go(strip, vidx0, eidx0, rows0, sem0)
    gather_go(strip + K, vidx1, eidx1, rows1, sem1)
    gather_wait(vidx0, eidx0, rows0, sem0)
    scat(rows0, vidx0, eidx0, ssem0)
    gather_wait(vidx1, eidx1, rows1, sem1)
    scat(rows1, vidx1, eidx1, ssem1)

    @pl.loop(1, NCHUNK // 2)
    def _(u):
        a = strip + (2 * u) * K
        scat_wait(rows0, vidx0, eidx0, ssem0)
        gather_go(a, vidx0, eidx0, rows0, sem0)
        scat_wait(rows1, vidx1, eidx1, ssem1)
        gather_go(a + K, vidx1, eidx1, rows1, sem1)
        gather_wait(vidx0, eidx0, rows0, sem0)
        scat(rows0, vidx0, eidx0, ssem0)
        gather_wait(vidx1, eidx1, rows1, sem1)
        scat(rows1, vidx1, eidx1, ssem1)

    scat_wait(rows0, vidx0, eidx0, ssem0)
    scat_wait(rows1, vidx1, eidx1, ssem1)


def _sc_body(xp_hbm, v_hbm, e_hbm, z_hbm, o_hbm, xv_out, xe_out,
             vidx0, eidx0, rows0, vidx1, eidx1, rows1, zblk, nbuf, sbuf, ones,
             cnt_sh, esum_sh, xv_sh, sem0, sem1, ssem0, ssem1, zsem):
    c = lax.axis_index("c")
    s = lax.axis_index("s")
    bufs = (vidx0, eidx0, rows0, sem0, ssem0,
            vidx1, eidx1, rows1, sem1, ssem1)

    pltpu.sync_copy(z_hbm, zblk)
    pltpu.sync_copy(o_hbm, ones)
    _zero_issue(zblk, esum_sh, s * EROWS, EROWS, zsem)
    _zero_issue(zblk.at[:, pl.ds(0, L)], cnt_sh, s * EROWS, EROWS, zsem)
    _zero_wait(zblk, esum_sh, s * EROWS, EROWS, zsem)
    _zero_wait(zblk.at[:, pl.ds(0, L)], cnt_sh, s * EROWS, EROWS, zsem)
    plsc.subcore_barrier()

    for cs in range(NPASS):
        q = c * NPASS + cs

        _zero_issue(zblk, xv_sh, s * VROWS, VROWS, zsem)

        if cs == 0:
            def scat_a(rows, vb, eb, ssem):
                pltpu.async_copy(rows, esum_sh.at[eb], ssem, add=True)
                pltpu.async_copy(ones, cnt_sh.at[eb], ssem, add=True)

            def scat_a_wait(rows, vb, eb, ssem):
                pltpu.make_async_copy(rows, esum_sh.at[eb], ssem).wait()
                pltpu.make_async_copy(ones, cnt_sh.at[eb], ssem).wait()
        else:
            def scat_a(rows, vb, eb, ssem):
                pltpu.async_copy(rows, esum_sh.at[eb], ssem, add=True)

            def scat_a_wait(rows, vb, eb, ssem):
                pltpu.make_async_copy(rows, esum_sh.at[eb], ssem).wait()
        _gs_pipeline(s, v_hbm, e_hbm,
                     lambda vb, eb: xp_hbm.at[q].at[vb], scat_a, scat_a_wait,
                     bufs)

        _zero_wait(zblk, xv_sh, s * VROWS, VROWS, zsem)
        plsc.subcore_barrier()

        row0 = s * EROWS
        done = 0
        while done < EROWS:
            nb = min(ZROWS, EROWS - done)
            off = row0 + done
            pltpu.sync_copy(esum_sh.at[pl.ds(off, nb)], nbuf.at[pl.ds(0, nb)])
            pltpu.sync_copy(cnt_sh.at[pl.ds(off, nb)], sbuf.at[pl.ds(0, nb)])

            @pl.loop(0, nb)
            def _(r):
                scale = 1.0 / jnp.maximum(sbuf[r, pl.ds(0, L)], 1.0)
                for j in range(CH // L):
                    nbuf[r, pl.ds(j * L, L)] = nbuf[r, pl.ds(j * L, L)] * scale

            pltpu.sync_copy(nbuf.at[pl.ds(0, nb)], xe_out.at[q].at[pl.ds(off, nb)])
            done += nb
        if cs + 1 < NPASS:
            _zero_issue(zblk, esum_sh, s * EROWS, EROWS, zsem)
        plsc.subcore_barrier()

        def scat_c(rows, vb, eb, ssem):
            pltpu.async_copy(rows, xv_sh.at[vb], ssem, add=True)

        def scat_c_wait(rows, vb, eb, ssem):
            pltpu.make_async_copy(rows, xv_sh.at[vb], ssem).wait()

        _gs_pipeline(s, v_hbm, e_hbm,
                     lambda vb, eb: xe_out.at[q].at[eb], scat_c,
                     scat_c_wait, bufs)

        if cs + 1 < NPASS:
            _zero_wait(zblk, esum_sh, s * EROWS, EROWS, zsem)
        plsc.subcore_barrier()

        pltpu.sync_copy(xv_sh.at[pl.ds(s * VROWS, VROWS)],
                        xv_out.at[q].at[pl.ds(s * VROWS, VROWS)])


_sc_sparse = pl.kernel(
    _sc_body,
    out_type=[jax.ShapeDtypeStruct((NQ, N_NODES, CH), jnp.float32),
              jax.ShapeDtypeStruct((NQ, NH_PAD, CH), jnp.float32)],
    mesh=_mesh,
    scratch_types=[
        pltpu.VMEM((K,), jnp.int32),
        pltpu.VMEM((K,), jnp.int32),
        pltpu.VMEM((K, CH), jnp.float32),
        pltpu.VMEM((K,), jnp.int32),
        pltpu.VMEM((K,), jnp.int32),
        pltpu.VMEM((K, CH), jnp.float32),
        pltpu.VMEM((ZROWS, CH), jnp.float32),
        pltpu.VMEM((ZROWS, CH), jnp.float32),
        pltpu.VMEM((ZROWS, L), jnp.float32),
        pltpu.VMEM((K, L), jnp.float32),
        pltpu.VMEM_SHARED((NH_PAD, L), jnp.float32),
        pltpu.VMEM_SHARED((NH_PAD, CH), jnp.float32),
        pltpu.VMEM_SHARED((N_NODES, CH), jnp.float32),
        pltpu.SemaphoreType.DMA,
        pltpu.SemaphoreType.DMA,
        pltpu.SemaphoreType.DMA,
        pltpu.SemaphoreType.DMA,
        pltpu.SemaphoreType.DMA,
    ],
    compiler_params=_sc_params,
)


_RB = 2000
_PREC = None


def _mmf_body(xp_ref, xv_ref, w_ref, c_ref, o_ref):
    a = jnp.concatenate(
        [xp_ref[i] + xv_ref[i] for i in range(NQ)], axis=1)
    a = jnp.maximum(a, 0.0) + jnp.minimum(a, 0.0) * c_ref[...]
    r = jnp.dot(a, w_ref[...], preferred_element_type=jnp.float32,
                precision=_PREC)
    for i in range(NQ):
        o_ref[i] = r[:, i * CH:(i + 1) * CH]


def _mmf(xp, xv, w, slope):
    return pl.pallas_call(
        _mmf_body,
        grid=(N_NODES // _RB,),
        in_specs=[pl.BlockSpec((NQ, _RB, CH), lambda i: (0, i, 0)),
                  pl.BlockSpec((NQ, _RB, CH), lambda i: (0, i, 0)),
                  pl.BlockSpec((C, C), lambda i: (0, 0)),
                  pl.BlockSpec((1, 1), lambda i: (0, 0))],
        out_specs=pl.BlockSpec((NQ, _RB, CH), lambda i: (0, i, 0)),
        out_shape=jax.ShapeDtypeStruct((NQ, N_NODES, CH), jnp.float32),
    )(xp, xv, w, slope)


def _epi_body(xp_ref, xv_ref, o_ref):
    a = xp_ref[0] + xv_ref[0]
    o_ref[...] = jnp.where(a >= 0, a, 0.01 * a)


def _epi(xp, xv):
    return pl.pallas_call(
        _epi_body,
        grid=(N_NODES // _RB,),
        in_specs=[pl.BlockSpec((1, _RB, CH), lambda i: (0, i, 0)),
                  pl.BlockSpec((1, _RB, CH), lambda i: (0, i, 0))],
        out_specs=pl.BlockSpec((_RB, CH), lambda i: (i, 0)),
        out_shape=jax.ShapeDtypeStruct((N_NODES, CH), jnp.float32),
    )(xp, xv)


def kernel(X, V, E, H, W1, W2, W3):
    del H
    w1p = jnp.pad(W1, ((0, C - NFEAT), (0, 0)))
    w3p = jnp.pad(W3, ((0, 0), (0, C - C3)))
    ws = jnp.stack([w1p, W2, w3p])
    slopes = jnp.array([1.0, 0.01, 0.01], jnp.float32).reshape(3, 1, 1)
    zeros_blk = jnp.zeros((ZROWS, CH), jnp.float32)
    ones_blk = jnp.ones((K, L), jnp.float32)

    xp0 = jnp.concatenate(
        [X.reshape(N_NODES, 2, CH).transpose(1, 0, 2),
         jnp.zeros((2, N_NODES, CH), jnp.float32)])
    xv0 = jnp.zeros_like(xp0)

    def step(carry, xs):
        xp_prev, xv_prev = carry
        w, slope = xs
        xp = _mmf(xp_prev, xv_prev, w, slope)
        xv, _ = _sc_sparse(xp, V, E, zeros_blk, ones_blk)
        return (xp, xv), None

    (xp3, xv3), _ = lax.scan(step, (xp0, xv0), (ws, slopes))
    out = _epi(xp3, xv3)
    return out[:, :C3]


# speedup vs baseline: 10.3476x; 1.4644x over previous
import jax
import jax.numpy as jnp
from jax import lax
from jax.experimental import pallas as pl
from jax.experimental.pallas import tpu as pltpu
from jax.experimental.pallas import tpu_sc as plsc

N_NODES = 10000
NNZ = 320000
N_HEDGES = 5000
NFEAT = 128
C = 256
NQ = 4
CH = C // NQ
C3 = 40

NC, NS, L = 2, 16, 16
NPASS = NQ // NC
NH_PAD = 5008
PAIRS_PER_SUB = NNZ // NS
K = 400
NCHUNK = PAIRS_PER_SUB // K
ZROWS = 32
EROWS = NH_PAD // NS
VROWS = N_NODES // NS

_mesh = plsc.VectorSubcoreMesh(core_axis_name="c", subcore_axis_name="s")
_sc_params = pltpu.CompilerParams(use_tc_tiling_on_sc=False)


def _zero_issue(zsrc, dst_sh, row0, nrows, sem):
    done = 0
    while done < nrows:
        nb = min(ZROWS, nrows - done)
        pltpu.async_copy(zsrc.at[pl.ds(0, nb)], dst_sh.at[pl.ds(row0 + done, nb)], sem)
        done += nb


def _zero_wait(zsrc, dst_sh, row0, nrows, sem):
    done = 0
    while done < nrows:
        nb = min(ZROWS, nrows - done)
        pltpu.make_async_copy(zsrc.at[pl.ds(0, nb)],
                              dst_sh.at[pl.ds(row0 + done, nb)], sem).wait()
        done += nb


def _gs_pipeline(s, v_hbm, e_hbm, gather_src, scat, scat_wait, bufs):
    (vidx0, eidx0, rows0, sem0, ssem0,
     vidx1, eidx1, rows1, sem1, ssem1) = bufs
    strip = s * PAIRS_PER_SUB

    def load_idx(base, vb, eb):
        pltpu.sync_copy(v_hbm.at[pl.ds(base, K)], vb)
        pltpu.sync_copy(e_hbm.at[pl.ds(base, K)], eb)

    def gather_go(base, vb, eb, rows, gsem):
        load_idx(base, vb, eb)
        pltpu.async_copy(gather_src(vb, eb), rows, gsem)

    def gather_wait(vb, eb, rows, gsem):
        pltpu.make_async_copy(gather_src(vb, eb), rows, gsem).wait()

    gather_go(strip, vidx0, eidx0, rows0, sem0)
    gather_go(strip + K, vidx1, eidx1, rows1, sem1)
    gather_wait(vidx0, eidx0, rows0, sem0)
    scat(rows0, vidx0, eidx0, ssem0)
    gather_wait(vidx1, eidx1, rows1, sem1)
    scat(rows1, vidx1, eidx1, ssem1)

    @pl.loop(1, NCHUNK // 2)
    def _(u):
        a = strip + (2 * u) * K
        scat_wait(rows0, vidx0, eidx0, ssem0)
        gather_go(a, vidx0, eidx0, rows0, sem0)
        scat_wait(rows1, vidx1, eidx1, ssem1)
        gather_go(a + K, vidx1, eidx1, rows1, sem1)
        gather_wait(vidx0, eidx0, rows0, sem0)
        scat(rows0, vidx0, eidx0, ssem0)
        gather_wait(vidx1, eidx1, rows1, sem1)
        scat(rows1, vidx1, eidx1, ssem1)

    scat_wait(rows0, vidx0, eidx0, ssem0)
    scat_wait(rows1, vidx1, eidx1, ssem1)


def _sc_body(npass, xp_hbm, v_hbm, e_hbm, z_hbm, o_hbm, xv_out, xe_out,
             vidx0, eidx0, rows0, vidx1, eidx1, rows1, zblk, nbuf, sbuf, ones,
             cnt_sh, esum_sh, xv_sh, sem0, sem1, ssem0, ssem1, zsem):
    c = lax.axis_index("c")
    s = lax.axis_index("s")
    bufs = (vidx0, eidx0, rows0, sem0, ssem0,
            vidx1, eidx1, rows1, sem1, ssem1)

    pltpu.sync_copy(z_hbm, zblk)
    pltpu.sync_copy(o_hbm, ones)
    _zero_issue(zblk, esum_sh, s * EROWS, EROWS, zsem)
    _zero_issue(zblk.at[:, pl.ds(0, L)], cnt_sh, s * EROWS, EROWS, zsem)
    _zero_wait(zblk, esum_sh, s * EROWS, EROWS, zsem)
    _zero_wait(zblk.at[:, pl.ds(0, L)], cnt_sh, s * EROWS, EROWS, zsem)
    plsc.subcore_barrier()

    for cs in range(npass):
        q = c * npass + cs

        _zero_issue(zblk, xv_sh, s * VROWS, VROWS, zsem)

        if cs == 0:
            def scat_a(rows, vb, eb, ssem):
                pltpu.async_copy(rows, esum_sh.at[eb], ssem, add=True)
                pltpu.async_copy(ones, cnt_sh.at[eb], ssem, add=True)

            def scat_a_wait(rows, vb, eb, ssem):
                pltpu.make_async_copy(rows, esum_sh.at[eb], ssem).wait()
                pltpu.make_async_copy(ones, cnt_sh.at[eb], ssem).wait()
        else:
            def scat_a(rows, vb, eb, ssem):
                pltpu.async_copy(rows, esum_sh.at[eb], ssem, add=True)

            def scat_a_wait(rows, vb, eb, ssem):
                pltpu.make_async_copy(rows, esum_sh.at[eb], ssem).wait()
        _gs_pipeline(s, v_hbm, e_hbm,
                     lambda vb, eb: xp_hbm.at[q].at[vb], scat_a, scat_a_wait,
                     bufs)

        _zero_wait(zblk, xv_sh, s * VROWS, VROWS, zsem)
        plsc.subcore_barrier()

        row0 = s * EROWS
        done = 0
        while done < EROWS:
            nb = min(ZROWS, EROWS - done)
            off = row0 + done
            pltpu.sync_copy(esum_sh.at[pl.ds(off, nb)], nbuf.at[pl.ds(0, nb)])
            pltpu.sync_copy(cnt_sh.at[pl.ds(off, nb)], sbuf.at[pl.ds(0, nb)])

            @pl.loop(0, nb)
            def _(r):
                scale = 1.0 / jnp.maximum(sbuf[r, pl.ds(0, L)], 1.0)
                for j in range(CH // L):
                    nbuf[r, pl.ds(j * L, L)] = nbuf[r, pl.ds(j * L, L)] * scale

            pltpu.sync_copy(nbuf.at[pl.ds(0, nb)], xe_out.at[q].at[pl.ds(off, nb)])
            done += nb
        if cs + 1 < npass:
            _zero_issue(zblk, esum_sh, s * EROWS, EROWS, zsem)
        plsc.subcore_barrier()

        def scat_c(rows, vb, eb, ssem):
            pltpu.async_copy(rows, xv_sh.at[vb], ssem, add=True)

        def scat_c_wait(rows, vb, eb, ssem):
            pltpu.make_async_copy(rows, xv_sh.at[vb], ssem).wait()

        _gs_pipeline(s, v_hbm, e_hbm,
                     lambda vb, eb: xe_out.at[q].at[eb], scat_c,
                     scat_c_wait, bufs)

        if cs + 1 < npass:
            _zero_wait(zblk, esum_sh, s * EROWS, EROWS, zsem)
        plsc.subcore_barrier()

        pltpu.sync_copy(xv_sh.at[pl.ds(s * VROWS, VROWS)],
                        xv_out.at[q].at[pl.ds(s * VROWS, VROWS)])


def _make_sc(npass):
    import functools
    nq_local = NC * npass
    return pl.kernel(
        functools.partial(_sc_body, npass),
        out_type=[jax.ShapeDtypeStruct((nq_local, N_NODES, CH), jnp.float32),
                  jax.ShapeDtypeStruct((nq_local, NH_PAD, CH), jnp.float32)],
        mesh=_mesh,
        scratch_types=[
            pltpu.VMEM((K,), jnp.int32),
            pltpu.VMEM((K,), jnp.int32),
            pltpu.VMEM((K, CH), jnp.float32),
            pltpu.VMEM((K,), jnp.int32),
            pltpu.VMEM((K,), jnp.int32),
            pltpu.VMEM((K, CH), jnp.float32),
            pltpu.VMEM((ZROWS, CH), jnp.float32),
            pltpu.VMEM((ZROWS, CH), jnp.float32),
            pltpu.VMEM((ZROWS, L), jnp.float32),
            pltpu.VMEM((K, L), jnp.float32),
            pltpu.VMEM_SHARED((NH_PAD, L), jnp.float32),
            pltpu.VMEM_SHARED((NH_PAD, CH), jnp.float32),
            pltpu.VMEM_SHARED((N_NODES, CH), jnp.float32),
            pltpu.SemaphoreType.DMA,
            pltpu.SemaphoreType.DMA,
            pltpu.SemaphoreType.DMA,
            pltpu.SemaphoreType.DMA,
            pltpu.SemaphoreType.DMA,
        ],
        compiler_params=_sc_params,
    )


_RB = 2000
_PREC = None


def _mmf_body(nq_out, xp_ref, xv_ref, w_ref, c_ref, o_ref):
    a = jnp.concatenate(
        [xp_ref[i] + xv_ref[i] for i in range(NQ)], axis=1)
    a = jnp.maximum(a, 0.0) + jnp.minimum(a, 0.0) * c_ref[...]
    r = jnp.dot(a, w_ref[...], preferred_element_type=jnp.float32,
                precision=_PREC)
    for i in range(nq_out):
        o_ref[i] = r[:, i * CH:(i + 1) * CH]


def _mmf(xp, xv, w, slope):
    import functools
    cols = w.shape[1]
    nq_out = cols // CH
    return pl.pallas_call(
        functools.partial(_mmf_body, nq_out),
        grid=(N_NODES // _RB,),
        in_specs=[pl.BlockSpec((NQ, _RB, CH), lambda i: (0, i, 0)),
                  pl.BlockSpec((NQ, _RB, CH), lambda i: (0, i, 0)),
                  pl.BlockSpec((C, cols), lambda i: (0, 0)),
                  pl.BlockSpec((1, 1), lambda i: (0, 0))],
        out_specs=pl.BlockSpec((nq_out, _RB, CH), lambda i: (0, i, 0)),
        out_shape=jax.ShapeDtypeStruct((nq_out, N_NODES, CH), jnp.float32),
    )(xp, xv, w, slope)


def _epi_body(xp_ref, xv_ref, o_ref):
    a = xp_ref[0] + xv_ref[0]
    o_ref[...] = jnp.where(a >= 0, a, 0.01 * a)


def _epi(xp, xv):
    nrows = xp.shape[1]
    rb = 1000
    return pl.pallas_call(
        _epi_body,
        grid=(nrows // rb,),
        in_specs=[pl.BlockSpec((1, rb, CH), lambda i: (0, i, 0)),
                  pl.BlockSpec((1, rb, CH), lambda i: (0, i, 0))],
        out_specs=pl.BlockSpec((rb, CH), lambda i: (i, 0)),
        out_shape=jax.ShapeDtypeStruct((nrows, CH), jnp.float32),
    )(xp, xv)


def kernel(X, V, E, H, W1, W2, W3):
    del H
    w1p = jnp.pad(W1, ((0, C - NFEAT), (0, 0)))
    w3p = jnp.pad(W3, ((0, 0), (0, C - C3)))
    ws = jnp.stack([w1p, W2, w3p])
    slopes = jnp.array([1.0, 0.01, 0.01], jnp.float32).reshape(3, 1, 1)
    zeros_blk = jnp.zeros((ZROWS, CH), jnp.float32)
    ones_blk = jnp.ones((K, L), jnp.float32)

    xp0 = jnp.concatenate(
        [X.reshape(N_NODES, 2, CH).transpose(1, 0, 2),
         jnp.zeros((2, N_NODES, CH), jnp.float32)])
    xv0 = jnp.zeros_like(xp0)

    if jax.device_count() >= 2:
        sc1 = _make_sc(1)
        mesh = jax.make_mesh((2,), ("d",))
        P = jax.sharding.PartitionSpec

        def sharded(X_, V_, E_, ws_, slopes_, z_, o_, xp0_, xv0_):
            d = lax.axis_index("d")

            def step(carry, xs):
                xp_f, xv_f = carry
                w, slope = xs
                w_h = lax.dynamic_slice_in_dim(w, d * (C // 2), C // 2, axis=1)
                xp_h = _mmf(xp_f, xv_f, w_h, slope)
                xv_h, _ = sc1(xp_h, V_, E_, z_, o_)
                xp_f = lax.all_gather(xp_h, "d", axis=0, tiled=True)
                xv_f = lax.all_gather(xv_h, "d", axis=0, tiled=True)
                return (xp_f, xv_f), None

            (xp3, xv3), _ = lax.scan(step, (xp0_, xv0_), (ws_, slopes_))
            half = N_NODES // 2
            xp_l = lax.dynamic_slice(xp3, (0, d * half, 0), (1, half, CH))
            xv_l = lax.dynamic_slice(xv3, (0, d * half, 0), (1, half, CH))
            return _epi(xp_l, xv_l)

        out2 = jax.shard_map(
            sharded,
            mesh=mesh,
            in_specs=(P(), P(), P(), P(), P(),
                      P(), P(), P(), P()),
            out_specs=P("d"),
            check_vma=False,
        )(X, V, E, ws, slopes, zeros_blk, ones_blk, xp0, xv0)
        out = out2
    else:
        sc2 = _make_sc(2)

        def step(carry, xs):
            xp_prev, xv_prev = carry
            w, slope = xs
            xp = _mmf(xp_prev, xv_prev, w, slope)
            xv, _ = sc2(xp, V, E, zeros_blk, ones_blk)
            return (xp, xv), None

        (xp3, xv3), _ = lax.scan(step, (xp0, xv0), (ws, slopes))
        out = _epi(xp3, xv3)
    return out[:, :C3]


# speedup vs baseline: 10.3737x; 1.0025x over previous
import functools

import jax
import jax.numpy as jnp
from jax import lax
from jax.experimental import pallas as pl
from jax.experimental.pallas import tpu as pltpu
from jax.experimental.pallas import tpu_sc as plsc

N_NODES = 10000
NNZ = 320000
N_HEDGES = 5000
NFEAT = 128
C = 256
NQ = 4
CH = C // NQ
C3 = 40

NC, NS, L = 2, 16, 16
NH_PAD = 5008
PAIRS_PER_SUB = NNZ // NS
K = 400
NCHUNK = PAIRS_PER_SUB // K
ZROWS = 32
EROWS = NH_PAD // NS
VROWS = N_NODES // NS

_mesh = plsc.VectorSubcoreMesh(core_axis_name="c", subcore_axis_name="s")
_sc_params = pltpu.CompilerParams(use_tc_tiling_on_sc=False)


def _zero_issue(zsrc, dst_sh, row0, nrows, sem):
    done = 0
    while done < nrows:
        nb = min(ZROWS, nrows - done)
        pltpu.async_copy(zsrc.at[pl.ds(0, nb)], dst_sh.at[pl.ds(row0 + done, nb)], sem)
        done += nb


def _zero_wait(zsrc, dst_sh, row0, nrows, sem):
    done = 0
    while done < nrows:
        nb = min(ZROWS, nrows - done)
        pltpu.make_async_copy(zsrc.at[pl.ds(0, nb)],
                              dst_sh.at[pl.ds(row0 + done, nb)], sem).wait()
        done += nb


def _gs_pipeline(s, v_hbm, e_hbm, gather_src, scat, scat_wait, bufs):
    (vidx0, eidx0, rows0, sem0, ssem0,
     vidx1, eidx1, rows1, sem1, ssem1) = bufs
    strip = s * PAIRS_PER_SUB

    def load_idx(base, vb, eb):
        pltpu.sync_copy(v_hbm.at[pl.ds(base, K)], vb)
        pltpu.sync_copy(e_hbm.at[pl.ds(base, K)], eb)

    def gather_go(base, vb, eb, rows, gsem):
        load_idx(base, vb, eb)
        pltpu.async_copy(gather_src(vb, eb), rows, gsem)

    def gather_wait(vb, eb, rows, gsem):
        pltpu.make_async_copy(gather_src(vb, eb), rows, gsem).wait()

    gather_go(strip, vidx0, eidx0, rows0, sem0)
    gather_go(strip + K, vidx1, eidx1, rows1, sem1)
    gather_wait(vidx0, eidx0, rows0, sem0)
    scat(rows0, vidx0, eidx0, ssem0)
    gather_wait(vidx1, eidx1, rows1, sem1)
    scat(rows1, vidx1, eidx1, ssem1)

    @pl.loop(1, NCHUNK // 2)
    def _(u):
        a = strip + (2 * u) * K
        scat_wait(rows0, vidx0, eidx0, ssem0)
        gather_go(a, vidx0, eidx0, rows0, sem0)
        scat_wait(rows1, vidx1, eidx1, ssem1)
        gather_go(a + K, vidx1, eidx1, rows1, sem1)
        gather_wait(vidx0, eidx0, rows0, sem0)
        scat(rows0, vidx0, eidx0, ssem0)
        gather_wait(vidx1, eidx1, rows1, sem1)
        scat(rows1, vidx1, eidx1, ssem1)

    scat_wait(rows0, vidx0, eidx0, ssem0)
    scat_wait(rows1, vidx1, eidx1, ssem1)


def _sc_body(npass, xp_hbm, v_hbm, e_hbm, z_hbm, o_hbm, xv_out, xe_out,
             vidx0, eidx0, rows0, vidx1, eidx1, rows1, zblk, nbuf, sbuf, ones,
             cnt_sh, esum_sh, xv_sh, sem0, sem1, ssem0, ssem1, zsem):
    c = lax.axis_index("c")
    s = lax.axis_index("s")
    bufs = (vidx0, eidx0, rows0, sem0, ssem0,
            vidx1, eidx1, rows1, sem1, ssem1)

    pltpu.sync_copy(z_hbm, zblk)
    pltpu.sync_copy(o_hbm, ones)
    _zero_issue(zblk, esum_sh, s * EROWS, EROWS, zsem)
    _zero_issue(zblk.at[:, pl.ds(0, L)], cnt_sh, s * EROWS, EROWS, zsem)
    _zero_wait(zblk, esum_sh, s * EROWS, EROWS, zsem)
    _zero_wait(zblk.at[:, pl.ds(0, L)], cnt_sh, s * EROWS, EROWS, zsem)
    plsc.subcore_barrier()

    for cs in range(npass):
        q = c * npass + cs

        _zero_issue(zblk, xv_sh, s * VROWS, VROWS, zsem)

        if cs == 0:
            def scat_a(rows, vb, eb, ssem):
                pltpu.async_copy(rows, esum_sh.at[eb], ssem, add=True)
                pltpu.async_copy(ones, cnt_sh.at[eb], ssem, add=True)

            def scat_a_wait(rows, vb, eb, ssem):
                pltpu.make_async_copy(rows, esum_sh.at[eb], ssem).wait()
                pltpu.make_async_copy(ones, cnt_sh.at[eb], ssem).wait()
        else:
            def scat_a(rows, vb, eb, ssem):
                pltpu.async_copy(rows, esum_sh.at[eb], ssem, add=True)

            def scat_a_wait(rows, vb, eb, ssem):
                pltpu.make_async_copy(rows, esum_sh.at[eb], ssem).wait()
        _gs_pipeline(s, v_hbm, e_hbm,
                     lambda vb, eb: xp_hbm.at[q].at[vb], scat_a, scat_a_wait,
                     bufs)

        _zero_wait(zblk, xv_sh, s * VROWS, VROWS, zsem)
        plsc.subcore_barrier()

        row0 = s * EROWS
        done = 0
        while done < EROWS:
            nb = min(ZROWS, EROWS - done)
            off = row0 + done
            pltpu.sync_copy(esum_sh.at[pl.ds(off, nb)], nbuf.at[pl.ds(0, nb)])
            pltpu.sync_copy(cnt_sh.at[pl.ds(off, nb)], sbuf.at[pl.ds(0, nb)])

            @pl.loop(0, nb)
            def _(r):
                scale = 1.0 / jnp.maximum(sbuf[r, pl.ds(0, L)], 1.0)
                for j in range(CH // L):
                    nbuf[r, pl.ds(j * L, L)] = nbuf[r, pl.ds(j * L, L)] * scale

            pltpu.sync_copy(nbuf.at[pl.ds(0, nb)], xe_out.at[q].at[pl.ds(off, nb)])
            done += nb
        if cs + 1 < npass:
            _zero_issue(zblk, esum_sh, s * EROWS, EROWS, zsem)
        plsc.subcore_barrier()

        def scat_c(rows, vb, eb, ssem):
            pltpu.async_copy(rows, xv_sh.at[vb], ssem, add=True)

        def scat_c_wait(rows, vb, eb, ssem):
            pltpu.make_async_copy(rows, xv_sh.at[vb], ssem).wait()

        _gs_pipeline(s, v_hbm, e_hbm,
                     lambda vb, eb: xe_out.at[q].at[eb], scat_c,
                     scat_c_wait, bufs)

        if cs + 1 < npass:
            _zero_wait(zblk, esum_sh, s * EROWS, EROWS, zsem)
        plsc.subcore_barrier()

        pltpu.sync_copy(xv_sh.at[pl.ds(s * VROWS, VROWS)],
                        xv_out.at[q].at[pl.ds(s * VROWS, VROWS)])


def _make_sc(npass):
    nq_local = NC * npass
    return pl.kernel(
        functools.partial(_sc_body, npass),
        out_type=[jax.ShapeDtypeStruct((nq_local, N_NODES, CH), jnp.float32),
                  jax.ShapeDtypeStruct((nq_local, NH_PAD, CH), jnp.float32)],
        mesh=_mesh,
        scratch_types=[
            pltpu.VMEM((K,), jnp.int32),
            pltpu.VMEM((K,), jnp.int32),
            pltpu.VMEM((K, CH), jnp.float32),
            pltpu.VMEM((K,), jnp.int32),
            pltpu.VMEM((K,), jnp.int32),
            pltpu.VMEM((K, CH), jnp.float32),
            pltpu.VMEM((ZROWS, CH), jnp.float32),
            pltpu.VMEM((ZROWS, CH), jnp.float32),
            pltpu.VMEM((ZROWS, L), jnp.float32),
            pltpu.VMEM((K, L), jnp.float32),
            pltpu.VMEM_SHARED((NH_PAD, L), jnp.float32),
            pltpu.VMEM_SHARED((NH_PAD, CH), jnp.float32),
            pltpu.VMEM_SHARED((N_NODES, CH), jnp.float32),
            pltpu.SemaphoreType.DMA,
            pltpu.SemaphoreType.DMA,
            pltpu.SemaphoreType.DMA,
            pltpu.SemaphoreType.DMA,
            pltpu.SemaphoreType.DMA,
        ],
        compiler_params=_sc_params,
    )


_RB = 2000
_PREC = None


def _mmf_body(nq_out, xp_ref, xv_ref, w_ref, c_ref, o_ref):
    a = jnp.concatenate(
        [xp_ref[i] + xv_ref[i] for i in range(NQ)], axis=1)
    a = jnp.maximum(a, 0.0) + jnp.minimum(a, 0.0) * c_ref[...]
    r = jnp.dot(a, w_ref[...], preferred_element_type=jnp.float32,
                precision=_PREC)
    for i in range(nq_out):
        o_ref[i] = r[:, i * CH:(i + 1) * CH]


def _mmf(xp, xv, w, slope):
    cols = w.shape[1]
    nq_out = cols // CH
    return pl.pallas_call(
        functools.partial(_mmf_body, nq_out),
        grid=(N_NODES // _RB,),
        in_specs=[pl.BlockSpec((NQ, _RB, CH), lambda i: (0, i, 0)),
                  pl.BlockSpec((NQ, _RB, CH), lambda i: (0, i, 0)),
                  pl.BlockSpec((C, cols), lambda i: (0, 0)),
                  pl.BlockSpec((1, 1), lambda i: (0, 0))],
        out_specs=pl.BlockSpec((nq_out, _RB, CH), lambda i: (0, i, 0)),
        out_shape=jax.ShapeDtypeStruct((nq_out, N_NODES, CH), jnp.float32),
    )(xp, xv, w, slope)


def _epi_body(xp_ref, xv_ref, o_ref):
    a = xp_ref[0] + xv_ref[0]
    o_ref[...] = jnp.where(a >= 0, a, 0.01 * a)


def _epi(xp, xv):
    nrows = xp.shape[1]
    rb = 1000
    return pl.pallas_call(
        _epi_body,
        grid=(nrows // rb,),
        in_specs=[pl.BlockSpec((1, rb, CH), lambda i: (0, i, 0)),
                  pl.BlockSpec((1, rb, CH), lambda i: (0, i, 0))],
        out_specs=pl.BlockSpec((rb, CH), lambda i: (i, 0)),
        out_shape=jax.ShapeDtypeStruct((nrows, CH), jnp.float32),
    )(xp, xv)


def kernel(X, V, E, H, W1, W2, W3):
    del H
    w1p = jnp.pad(W1, ((0, C - NFEAT), (0, 0)))
    w3p = jnp.pad(W3, ((0, 0), (0, C - C3)))
    ws = jnp.stack([w1p, W2, w3p])
    slopes = jnp.array([1.0, 0.01, 0.01], jnp.float32).reshape(3, 1, 1)
    zeros_blk = jnp.zeros((ZROWS, CH), jnp.float32)
    ones_blk = jnp.ones((K, L), jnp.float32)

    xp0 = jnp.concatenate(
        [X.reshape(N_NODES, 2, CH).transpose(1, 0, 2),
         jnp.zeros((2, N_NODES, CH), jnp.float32)])
    xv0 = jnp.zeros_like(xp0)

    if jax.device_count() >= 2:
        sc1 = _make_sc(1)
        mesh = jax.make_mesh((2,), ("d",))
        P = jax.sharding.PartitionSpec

        def sharded(X_, V_, E_, ws_, slopes_, z_, o_, xp0_, xv0_):
            d = lax.axis_index("d")

            def step(carry, xs):
                xp_f, xv_f = carry
                w, slope = xs
                w_h = lax.dynamic_slice_in_dim(w, d * (C // 2), C // 2, axis=1)
                xp_h = _mmf(xp_f, xv_f, w_h, slope)
                xv_h, _ = sc1(xp_h, V_, E_, z_, o_)
                xp_f = lax.all_gather(xp_h, "d", axis=0, tiled=True)
                xv_f = lax.all_gather(xv_h, "d", axis=0, tiled=True)
                return (xp_f, xv_f), None

            (xp3, xv3), _ = lax.scan(step, (xp0_, xv0_), (ws_, slopes_))
            half = N_NODES // 2
            xp_l = lax.dynamic_slice(xp3, (0, d * half, 0), (1, half, CH))
            xv_l = lax.dynamic_slice(xv3, (0, d * half, 0), (1, half, CH))
            return _epi(xp_l, xv_l)

        out2 = jax.shard_map(
            sharded,
            mesh=mesh,
            in_specs=(P(), P(), P(), P(), P(),
                      P(), P(), P(), P()),
            out_specs=P("d"),
            check_vma=False,
        )(X, V, E, ws, slopes, zeros_blk, ones_blk, xp0, xv0)
        out = out2
    else:
        sc2 = _make_sc(2)

        def step(carry, xs):
            xp_prev, xv_prev = carry
            w, slope = xs
            xp = _mmf(xp_prev, xv_prev, w, slope)
            xv, _ = sc2(xp, V, E, zeros_blk, ones_blk)
            return (xp, xv), None

        (xp3, xv3), _ = lax.scan(step, (xp0, xv0), (ws, slopes))
        out = _epi(xp3, xv3)
    return out[:, :C3]


# speedup vs baseline: 11.1292x; 1.0728x over previous
import functools

import jax
import jax.numpy as jnp
from jax import lax
from jax.experimental import pallas as pl
from jax.experimental.pallas import tpu as pltpu
from jax.experimental.pallas import tpu_sc as plsc

N_NODES = 10000
NNZ = 320000
N_HEDGES = 5000
NFEAT = 128
C = 256
NQ = 4
CH = C // NQ
C3 = 40

NC, NS, L = 2, 16, 16
NH_PAD = 5008
PAIRS_PER_SUB = NNZ // NS
K = 400
NCHUNK = PAIRS_PER_SUB // K
ZROWS = 32
EROWS = NH_PAD // NS
VROWS = N_NODES // NS

_mesh = plsc.VectorSubcoreMesh(core_axis_name="c", subcore_axis_name="s")
_sc_params = pltpu.CompilerParams(use_tc_tiling_on_sc=False)


def _zero_issue(zsrc, dst_sh, row0, nrows, sem):
    done = 0
    while done < nrows:
        nb = min(ZROWS, nrows - done)
        pltpu.async_copy(zsrc.at[pl.ds(0, nb)], dst_sh.at[pl.ds(row0 + done, nb)], sem)
        done += nb


def _zero_wait(zsrc, dst_sh, row0, nrows, sem):
    done = 0
    while done < nrows:
        nb = min(ZROWS, nrows - done)
        pltpu.make_async_copy(zsrc.at[pl.ds(0, nb)],
                              dst_sh.at[pl.ds(row0 + done, nb)], sem).wait()
        done += nb


def _gs_pipeline(s, v_hbm, e_hbm, gather_src, scat, scat_wait, bufs):
    (vidx0, eidx0, rows0, sem0, ssem0,
     vidx1, eidx1, rows1, sem1, ssem1) = bufs
    strip = s * PAIRS_PER_SUB

    def load_idx(base, vb, eb):
        pltpu.sync_copy(v_hbm.at[pl.ds(base, K)], vb)
        pltpu.sync_copy(e_hbm.at[pl.ds(base, K)], eb)

    def gather_go(base, vb, eb, rows, gsem):
        load_idx(base, vb, eb)
        pltpu.async_copy(gather_src(vb, eb), rows, gsem)

    def gather_wait(vb, eb, rows, gsem):
        pltpu.make_async_copy(gather_src(vb, eb), rows, gsem).wait()

    gather_go(strip, vidx0, eidx0, rows0, sem0)
    gather_go(strip + K, vidx1, eidx1, rows1, sem1)
    gather_wait(vidx0, eidx0, rows0, sem0)
    scat(rows0, vidx0, eidx0, ssem0)
    gather_wait(vidx1, eidx1, rows1, sem1)
    scat(rows1, vidx1, eidx1, ssem1)

    @pl.loop(1, NCHUNK // 2)
    def _(u):
        a = strip + (2 * u) * K
        scat_wait(rows0, vidx0, eidx0, ssem0)
        gather_go(a, vidx0, eidx0, rows0, sem0)
        scat_wait(rows1, vidx1, eidx1, ssem1)
        gather_go(a + K, vidx1, eidx1, rows1, sem1)
        gather_wait(vidx0, eidx0, rows0, sem0)
        scat(rows0, vidx0, eidx0, ssem0)
        gather_wait(vidx1, eidx1, rows1, sem1)
        scat(rows1, vidx1, eidx1, ssem1)

    scat_wait(rows0, vidx0, eidx0, ssem0)
    scat_wait(rows1, vidx1, eidx1, ssem1)


def _sc_body(npass, xp_hbm, v_hbm, e_hbm, z_hbm, o_hbm, xv_out, xe_out,
             vidx0, eidx0, rows0, vidx1, eidx1, rows1, zblk, nbuf, sbuf, ones,
             cnt_sh, esum_sh, xv_sh, sem0, sem1, ssem0, ssem1, zsem):
    c = lax.axis_index("c")
    s = lax.axis_index("s")
    bufs = (vidx0, eidx0, rows0, sem0, ssem0,
            vidx1, eidx1, rows1, sem1, ssem1)

    pltpu.sync_copy(z_hbm, zblk)
    pltpu.sync_copy(o_hbm, ones)
    _zero_issue(zblk, esum_sh, s * EROWS, EROWS, zsem)
    _zero_issue(zblk.at[:, pl.ds(0, L)], cnt_sh, s * EROWS, EROWS, zsem)
    _zero_wait(zblk, esum_sh, s * EROWS, EROWS, zsem)
    _zero_wait(zblk.at[:, pl.ds(0, L)], cnt_sh, s * EROWS, EROWS, zsem)
    plsc.subcore_barrier()

    for cs in range(npass):
        q = c * npass + cs

        _zero_issue(zblk, xv_sh, s * VROWS, VROWS, zsem)

        if cs == 0:
            def scat_a(rows, vb, eb, ssem):
                pltpu.async_copy(rows, esum_sh.at[eb], ssem, add=True)
                pltpu.async_copy(ones, cnt_sh.at[eb], ssem, add=True)

            def scat_a_wait(rows, vb, eb, ssem):
                pltpu.make_async_copy(rows, esum_sh.at[eb], ssem).wait()
                pltpu.make_async_copy(ones, cnt_sh.at[eb], ssem).wait()
        else:
            def scat_a(rows, vb, eb, ssem):
                pltpu.async_copy(rows, esum_sh.at[eb], ssem, add=True)

            def scat_a_wait(rows, vb, eb, ssem):
                pltpu.make_async_copy(rows, esum_sh.at[eb], ssem).wait()
        _gs_pipeline(s, v_hbm, e_hbm,
                     lambda vb, eb: xp_hbm.at[q].at[vb], scat_a, scat_a_wait,
                     bufs)

        _zero_wait(zblk, xv_sh, s * VROWS, VROWS, zsem)
        plsc.subcore_barrier()

        row0 = s * EROWS
        done = 0
        while done < EROWS:
            nb = min(ZROWS, EROWS - done)
            off = row0 + done
            pltpu.sync_copy(esum_sh.at[pl.ds(off, nb)], nbuf.at[pl.ds(0, nb)])
            pltpu.sync_copy(cnt_sh.at[pl.ds(off, nb)], sbuf.at[pl.ds(0, nb)])

            @pl.loop(0, nb)
            def _(r):
                scale = 1.0 / jnp.maximum(sbuf[r, pl.ds(0, L)], 1.0)
                for j in range(CH // L):
                    nbuf[r, pl.ds(j * L, L)] = nbuf[r, pl.ds(j * L, L)] * scale

            pltpu.sync_copy(nbuf.at[pl.ds(0, nb)], xe_out.at[q].at[pl.ds(off, nb)])
            done += nb
        if cs + 1 < npass:
            _zero_issue(zblk, esum_sh, s * EROWS, EROWS, zsem)
        plsc.subcore_barrier()

        def scat_c(rows, vb, eb, ssem):
            pltpu.async_copy(rows, xv_sh.at[vb], ssem, add=True)

        def scat_c_wait(rows, vb, eb, ssem):
            pltpu.make_async_copy(rows, xv_sh.at[vb], ssem).wait()

        _gs_pipeline(s, v_hbm, e_hbm,
                     lambda vb, eb: xe_out.at[q].at[eb], scat_c,
                     scat_c_wait, bufs)

        if cs + 1 < npass:
            _zero_wait(zblk, esum_sh, s * EROWS, EROWS, zsem)
        plsc.subcore_barrier()

        pltpu.sync_copy(xv_sh.at[pl.ds(s * VROWS, VROWS)],
                        xv_out.at[q].at[pl.ds(s * VROWS, VROWS)])


def _make_sc(npass):
    nq_local = NC * npass
    return pl.kernel(
        functools.partial(_sc_body, npass),
        out_type=[jax.ShapeDtypeStruct((nq_local, N_NODES, CH), jnp.float32),
                  jax.ShapeDtypeStruct((nq_local, NH_PAD, CH), jnp.float32)],
        mesh=_mesh,
        scratch_types=[
            pltpu.VMEM((K,), jnp.int32),
            pltpu.VMEM((K,), jnp.int32),
            pltpu.VMEM((K, CH), jnp.float32),
            pltpu.VMEM((K,), jnp.int32),
            pltpu.VMEM((K,), jnp.int32),
            pltpu.VMEM((K, CH), jnp.float32),
            pltpu.VMEM((ZROWS, CH), jnp.float32),
            pltpu.VMEM((ZROWS, CH), jnp.float32),
            pltpu.VMEM((ZROWS, L), jnp.float32),
            pltpu.VMEM((K, L), jnp.float32),
            pltpu.VMEM_SHARED((NH_PAD, L), jnp.float32),
            pltpu.VMEM_SHARED((NH_PAD, CH), jnp.float32),
            pltpu.VMEM_SHARED((N_NODES, CH), jnp.float32),
            pltpu.SemaphoreType.DMA,
            pltpu.SemaphoreType.DMA,
            pltpu.SemaphoreType.DMA,
            pltpu.SemaphoreType.DMA,
            pltpu.SemaphoreType.DMA,
        ],
        compiler_params=_sc_params,
    )


_RB = 2000
_PREC = None


def _mm_body(xp_ref, w_ref, o_ref):
    a = jnp.concatenate([xp_ref[i] for i in range(NQ)], axis=1)
    r = jnp.dot(a, w_ref[...], preferred_element_type=jnp.float32,
                precision=_PREC)
    for i in range(o_ref.shape[0]):
        o_ref[i] = r[:, i * CH:(i + 1) * CH]


def _mm(a, w):
    cols = w.shape[1]
    nq_out = cols // CH
    return pl.pallas_call(
        _mm_body,
        grid=(N_NODES // _RB,),
        in_specs=[pl.BlockSpec((NQ, _RB, CH), lambda i: (0, i, 0)),
                  pl.BlockSpec((C, cols), lambda i: (0, 0))],
        out_specs=pl.BlockSpec((nq_out, _RB, CH), lambda i: (0, i, 0)),
        out_shape=jax.ShapeDtypeStruct((nq_out, N_NODES, CH), jnp.float32),
    )(a, w)


def _addact_body(xp_ref, xv_ref, o_ref):
    a = xp_ref[...] + xv_ref[...]
    o_ref[...] = jnp.where(a >= 0, a, 0.01 * a)


def _addact(xp, xv):
    nq = xp.shape[0]
    return pl.pallas_call(
        _addact_body,
        grid=(N_NODES // _RB,),
        in_specs=[pl.BlockSpec((nq, _RB, CH), lambda i: (0, i, 0)),
                  pl.BlockSpec((nq, _RB, CH), lambda i: (0, i, 0))],
        out_specs=pl.BlockSpec((nq, _RB, CH), lambda i: (0, i, 0)),
        out_shape=jax.ShapeDtypeStruct((nq, N_NODES, CH), jnp.float32),
    )(xp, xv)


def kernel(X, V, E, H, W1, W2, W3):
    del H
    w1p = jnp.pad(W1, ((0, C - NFEAT), (0, 0)))
    w3p = jnp.pad(W3, ((0, 0), (0, C - C3)))
    ws = jnp.stack([w1p, W2, w3p])
    zeros_blk = jnp.zeros((ZROWS, CH), jnp.float32)
    ones_blk = jnp.ones((K, L), jnp.float32)

    a0 = jnp.concatenate(
        [X.reshape(N_NODES, 2, CH).transpose(1, 0, 2),
         jnp.zeros((2, N_NODES, CH), jnp.float32)])

    if jax.device_count() >= 2:
        sc1 = _make_sc(1)
        mesh = jax.make_mesh((2,), ("d",))
        P = jax.sharding.PartitionSpec

        def sharded(V_, E_, ws_, z_, o_, a0_):
            d = lax.axis_index("d")

            def step(a_f, w):
                w_h = lax.dynamic_slice_in_dim(w, d * (C // 2), C // 2, axis=1)
                xp_h = _mm(a_f, w_h)
                xv_h, _ = sc1(xp_h, V_, E_, z_, o_)
                a_h = _addact(xp_h, xv_h)
                return lax.all_gather(a_h, "d", axis=0, tiled=True), None

            a3, _ = lax.scan(step, a0_, ws_)
            half = N_NODES // 2
            return lax.dynamic_slice(a3, (0, d * half, 0),
                                     (1, half, CH))[0]

        out = jax.shard_map(
            sharded,
            mesh=mesh,
            in_specs=(P(), P(), P(), P(), P(), P()),
            out_specs=P("d"),
            check_vma=False,
        )(V, E, ws, zeros_blk, ones_blk, a0)
    else:
        sc2 = _make_sc(2)

        def step(a_prev, w):
            xp = _mm(a_prev, w)
            xv, _ = sc2(xp, V, E, zeros_blk, ones_blk)
            return _addact(xp, xv), None

        (a3), _ = lax.scan(step, a0, ws)
        out = a3[0]
    return out[:, :C3]
